# baseline (speedup 1.0000x reference)
import functools

import jax
import jax.numpy as jnp
from jax import lax
from jax.experimental import pallas as pl
from jax.experimental.pallas import tpu as pltpu

_NUM_CLASSES = 6
_TOP_K = 100
_OVERLAP_THRESH = 0.5
_CONF_THRESH = 0.01
_NMS_THRESH = 0.45
_VAR0 = 0.1
_VAR1 = 0.2


def _point_form(priors):
    return jnp.concatenate(
        [priors[:, :2] - priors[:, 2:] / 2.0, priors[:, :2] + priors[:, 2:] / 2.0],
        axis=1)


def _jaccard(box_a, box_b):
    max_xy = jnp.minimum(box_a[:, None, 2:], box_b[None, :, 2:])
    min_xy = jnp.maximum(box_a[:, None, :2], box_b[None, :, :2])
    inter_wh = jnp.clip(max_xy - min_xy, 0.0, None)
    inter = inter_wh[..., 0] * inter_wh[..., 1]
    area_a = ((box_a[:, 2] - box_a[:, 0]) * (box_a[:, 3] - box_a[:, 1]))[:, None]
    area_b = ((box_b[:, 2] - box_b[:, 0]) * (box_b[:, 3] - box_b[:, 1]))[None, :]
    return inter / (area_a + area_b - inter)


def _encode(matched, priors):
    g_cxcy = (matched[:, :2] + matched[:, 2:]) / 2.0 - priors[:, :2]
    g_cxcy = g_cxcy / (_VAR0 * priors[:, 2:])
    g_wh = (matched[:, 2:] - matched[:, :2]) / priors[:, 2:]
    g_wh = jnp.log(g_wh) / _VAR1
    return jnp.concatenate([g_cxcy, g_wh], axis=1)


def _decode(loc, priors):
    boxes = jnp.concatenate(
        [priors[:, :2] + loc[:, :2] * _VAR0 * priors[:, 2:],
         priors[:, 2:] * jnp.exp(loc[:, 2:] * _VAR1)], axis=1)
    xy = boxes[:, :2] - boxes[:, 2:] / 2.0
    return jnp.concatenate([xy, xy + boxes[:, 2:]], axis=1)


def _match(truths, priors, labels):
    overlaps = _jaccard(truths, _point_form(priors))
    best_prior_idx = jnp.argmax(overlaps, axis=1)
    best_truth_overlap = jnp.max(overlaps, axis=0)
    best_truth_idx = jnp.argmax(overlaps, axis=0)
    best_truth_overlap = best_truth_overlap.at[best_prior_idx].set(2.0)
    best_truth_idx = best_truth_idx.at[best_prior_idx].set(
        jnp.arange(truths.shape[0]))
    matches_ = truths[best_truth_idx]
    conf = labels[best_truth_idx] + 1.0
    conf = jnp.where(best_truth_overlap < _OVERLAP_THRESH, 0.0, conf)
    loc = _encode(matches_, priors)
    return loc, conf


def _greedy_nms_keep(boxes, valid):
    iou = _jaccard(boxes, boxes)
    K = boxes.shape[0]
    keep = jnp.zeros((K,), dtype=bool)
    for i in range(K):
        sup = jnp.any(keep & (iou[:, i] > _NMS_THRESH))
        keep = keep.at[i].set(valid[i] & jnp.logical_not(sup))
    return keep


def _core(loc_data, conf_data, prior_data, targets):
    batch = loc_data.shape[0]
    num_priors = loc_data.shape[1]
    C = _NUM_CLASSES
    loc_t_list, conf_t_list = [], []
    for idx in range(batch):
        l, c = _match(targets[idx][:, :-1], prior_data, targets[idx][:, -1])
        loc_t_list.append(l)
        conf_t_list.append(c)
    loc_t = jnp.stack(loc_t_list)
    conf_t = jnp.stack(conf_t_list)
    conf3 = conf_data.reshape(batch, num_priors, C)
    conf_preds = jnp.transpose(conf3, (0, 2, 1))
    results = []
    for i in range(batch):
        decoded = _decode(loc_data[i], prior_data)
        rows_list, valid_list = [], []
        for cl in range(1, C):
            scores = conf_preds[i, cl]
            masked = jnp.where(scores > _CONF_THRESH, scores, -jnp.inf)
            _, cand_idx = jax.lax.top_k(jax.lax.stop_gradient(masked), _TOP_K)
            cand_scores = scores[cand_idx]
            cand_valid = cand_scores > _CONF_THRESH
            boxes_c = decoded[cand_idx]
            keep = _greedy_nms_keep(jax.lax.stop_gradient(boxes_c), cand_valid)
            row = jnp.concatenate(
                [cand_scores[:, None], boxes_c, loc_data[i][cand_idx],
                 conf3[i][cand_idx], loc_t[i][cand_idx],
                 conf_t[i][cand_idx][:, None]], axis=1)
            rows_list.append(row)
            valid_list.append(keep)
        rows = jnp.concatenate(rows_list, axis=0)
        valid = jnp.concatenate(valid_list, axis=0)
        sort_conf = rows[:, 0]
        rows = rows.at[:, 0].set(float(i))
        M = rows.shape[0]
        eq = jnp.all(rows[:, None, :] == rows[None, :, :], axis=-1)
        earlier = jnp.arange(M)[None, :] < jnp.arange(M)[:, None]
        dup = jnp.any(eq & earlier & valid[None, :], axis=1)
        valid = valid & jnp.logical_not(dup)
        key_ = jnp.where(valid, sort_conf, -jnp.inf)
        order = jnp.argsort(-key_)[:_TOP_K]
        res_i = jnp.where(valid[order][:, None], rows[order], 0.0)
        results.append(res_i)
    result = jnp.stack(results)
    rois = result[..., 0:5]
    loc = result[..., 5:9]
    cls = result[..., 9:9 + C]
    loc_truth = result[..., 9 + C:13 + C]
    conf_truth = result[..., 13 + C:14 + C]
    return rois, loc, cls, loc_truth, conf_truth


def kernel(loc_data, conf_data, prior_data, targets):
    return _core(loc_data, conf_data, prior_data, targets)


# speedup vs baseline: 29.2405x; 29.2405x over previous
import functools
import struct

import jax
import jax.numpy as jnp
from jax import lax
from jax.experimental import pallas as pl
from jax.experimental.pallas import tpu as pltpu
from jax.experimental.pallas import tpu_sc as plsc

_C = 6
_NCL = _C - 1
_B = 2
_NP = 5000
_PADN = 5024
_TOPK = 100
_CAND = 112
_OVERLAP_THRESH = 0.5
_CONF_THRESH = 0.01
_NMS_THRESH = 0.45
_VAR0 = 0.1
_VAR1 = 0.2
_NOBJ = 8
_COLS = 24
_PCOLS = 128
_ZROW = _NP
_BITS001 = struct.unpack("<i", struct.pack("<f", _CONF_THRESH))[0]


def _tc_prep_body(locT_ref, confT_ref, priorsT_ref, targets_ref,
                  scores_ref, colT_ref):
    cx = priorsT_ref[0:1, :]
    cy = priorsT_ref[1:2, :]
    pw = priorsT_ref[2:3, :]
    ph = priorsT_ref[3:4, :]
    col = lax.broadcasted_iota(jnp.int32, (1, _PADN), 1)
    real = col < _NP
    realf = real.astype(jnp.float32)
    px1 = cx - pw * 0.5
    py1 = cy - ph * 0.5
    px2 = cx + pw * 0.5
    py2 = cy + ph * 0.5
    p_area = (px2 - px1) * (py2 - py1)

    for i in range(_B):
        tx1 = targets_ref[i, :, 0:1]
        ty1 = targets_ref[i, :, 1:2]
        tx2 = targets_ref[i, :, 2:3]
        ty2 = targets_ref[i, :, 3:4]
        tlab = targets_ref[i, :, 4:5]
        t_area = (tx2 - tx1) * (ty2 - ty1)
        ix1 = jnp.maximum(tx1, px1)
        iy1 = jnp.maximum(ty1, py1)
        ix2 = jnp.minimum(tx2, px2)
        iy2 = jnp.minimum(ty2, py2)
        iw = jnp.clip(ix2 - ix1, 0.0, None)
        ih = jnp.clip(iy2 - iy1, 0.0, None)
        inter = iw * ih
        ov = inter / (t_area + p_area - inter)
        ov = jnp.where(real, ov, -1.0)
        bigcol = jnp.where(ov == jnp.max(ov, axis=1, keepdims=True),
                           jnp.broadcast_to(col, ov.shape), _PADN)
        best_prior_idx = jnp.min(bigcol, axis=1, keepdims=True)
        trow = lax.broadcasted_iota(jnp.int32, (_NOBJ, 1), 0)
        ovmax0 = jnp.max(ov, axis=0, keepdims=True)
        bigrow = jnp.where(ov == ovmax0,
                           jnp.broadcast_to(trow, ov.shape), _NOBJ)
        best_truth_idx = jnp.min(bigrow, axis=0, keepdims=True)
        best_truth_overlap = ovmax0
        for t in range(_NOBJ):
            hit = col == best_prior_idx[t, 0]
            best_truth_overlap = jnp.where(hit, 2.0, best_truth_overlap)
            best_truth_idx = jnp.where(hit, t, best_truth_idx)
        mx1 = jnp.zeros((1, _PADN), jnp.float32)
        my1 = jnp.zeros((1, _PADN), jnp.float32)
        mx2 = jnp.zeros((1, _PADN), jnp.float32)
        my2 = jnp.zeros((1, _PADN), jnp.float32)
        mlab = jnp.zeros((1, _PADN), jnp.float32)
        for t in range(_NOBJ):
            sel = best_truth_idx == t
            mx1 = jnp.where(sel, tx1[t, 0], mx1)
            my1 = jnp.where(sel, ty1[t, 0], my1)
            mx2 = jnp.where(sel, tx2[t, 0], mx2)
            my2 = jnp.where(sel, ty2[t, 0], my2)
            mlab = jnp.where(sel, tlab[t, 0], mlab)
        conf_t = jnp.where(best_truth_overlap < _OVERLAP_THRESH, 0.0,
                           mlab + 1.0)
        g_cx = ((mx1 + mx2) * 0.5 - cx) / (_VAR0 * pw)
        g_cy = ((my1 + my2) * 0.5 - cy) / (_VAR0 * ph)
        safe_w = jnp.where(real, (mx2 - mx1) / pw, 1.0)
        safe_h = jnp.where(real, (my2 - my1) / ph, 1.0)
        g_w = jnp.log(safe_w) / _VAR1
        g_h = jnp.log(safe_h) / _VAR1

        l0 = locT_ref[i, 0:1, :]
        l1 = locT_ref[i, 1:2, :]
        l2 = locT_ref[i, 2:3, :]
        l3 = locT_ref[i, 3:4, :]
        dcx = cx + l0 * _VAR0 * pw
        dcy = cy + l1 * _VAR0 * ph
        dw = pw * jnp.exp(l2 * _VAR1)
        dh = ph * jnp.exp(l3 * _VAR1)
        dx1 = dcx - dw * 0.5
        dy1 = dcy - dh * 0.5
        dx2 = dx1 + dw
        dy2 = dy1 + dh

        for c in range(1, _C):
            srow = confT_ref[i, c:c + 1, :] * realf
            scores_ref[i * _NCL + (c - 1), :] = srow[0]

        cols = [jnp.full((1, _PADN), float(i), jnp.float32),
                dx1, dy1, dx2, dy2,
                l0, l1, l2, l3]
        for c in range(_C):
            cols.append(confT_ref[i, c:c + 1, :])
        cols += [g_cx, g_cy, g_w, g_h, conf_t]
        for k, v in enumerate(cols):
            colT_ref[i, k, :] = (v * realf)[0]
        for k in range(len(cols), _COLS):
            colT_ref[i, k, :] = jnp.zeros((_PADN,), jnp.float32)


def _tc_prep(loc_data, conf_data, prior_data, targets):
    locT = jnp.pad(jnp.transpose(loc_data, (0, 2, 1)),
                   ((0, 0), (0, 0), (0, _PADN - _NP)))
    conf3 = conf_data.reshape(_B, _NP, _C)
    confT = jnp.pad(jnp.transpose(conf3, (0, 2, 1)),
                    ((0, 0), (0, 0), (0, _PADN - _NP)))
    priorsT = jnp.pad(prior_data.T, ((0, 0), (0, _PADN - _NP)),
                      constant_values=1.0)
    padmask = jnp.arange(_PADN)[None, :] >= _NP
    priorsT = jnp.where(padmask & (jnp.arange(4)[:, None] < 2), 100.0, priorsT)
    scores, colT = pl.pallas_call(
        _tc_prep_body,
        out_shape=[
            jax.ShapeDtypeStruct((_B * _NCL, _PADN), jnp.float32),
            jax.ShapeDtypeStruct((_B, _COLS, _PADN), jnp.float32),
        ],
    )(locT, confT, priorsT, targets)
    payload = jnp.pad(jnp.transpose(colT, (0, 2, 1)),
                      ((0, 0), (0, 0), (0, _PCOLS - _COLS)))
    return scores, payload.reshape(_B * _PADN, _PCOLS)


_CBUF = 512
_CCAP = _CBUF - 16
_MRG = 1024
_NVEC = _PADN // 16


def _sc_select_body(scores_hbm, payload_hbm, out_hbm,
                    sv, cb_key, cb_idx, hist, cand_rows, bx, keep,
                    m_key, m_prior, m_keep, seen, outp, shared, sem):
    c = lax.axis_index("c")
    s = lax.axis_index("s")
    lane = lax.iota(jnp.int32, 16)
    ones16 = jnp.ones((16,), jnp.int32)
    zero16 = jnp.zeros((16,), jnp.int32)

    def hist_zero():
        def z(j, _):
            plsc.store_scatter(hist, [j * 16 + lane], zero16)
            return 0
        lax.fori_loop(0, 256, z, 0)

    def hist_pass(bucket_fn):
        def body(j, _):
            b, m = bucket_fn(j)
            plsc.addupdate_scatter(hist, [lane * 256 + b], ones16, mask=m)
            return 0
        lax.fori_loop(0, _NVEC, body, 0)

    def hist_select(rank):
        run = jnp.int32(0)
        B = jnp.int32(0)
        ca = jnp.int32(0)
        found = jnp.bool_(False)
        for g in range(15, -1, -1):
            acc = zero16
            for l in range(16):
                acc = acc + hist[pl.ds(l * 256 + g * 16, 16)]
            sfx = lax.rev(plsc.cumsum(lax.rev(acc, (0,))), (0,)) + run
            mask = sfx >= rank
            cnt = plsc.all_reduce_population_count(mask)[0]
            cav = jnp.sum(jnp.where(lane == cnt, sfx, 0))
            ca_g = jnp.where(cnt == 16, run, cav)
            hit = jnp.logical_and(jnp.logical_not(found), cnt > 0)
            B = jnp.where(hit, g * 16 + cnt - 1, B)
            ca = jnp.where(hit, ca_g, ca)
            found = jnp.logical_or(found, cnt > 0)
            run = run + jnp.sum(acc)
        return B, ca, found

    def bitonic(keyref, valref, n):
        nb = n // 16

        def vsort_sweep(kk):
            k16 = kk // 16
            def body(b, _):
                base = b * 16
                key = plsc.load_gather(keyref, [base + lane])
                val = plsc.load_gather(valref, [base + lane])
                desc = (b & k16) == 0
                tkey = jnp.where(desc, key, -1 - key)
                skey, sval = plsc.sort_key_val(tkey, val, descending=True)
                skey = jnp.where(desc, skey, -1 - skey)
                plsc.store_scatter(keyref, [base + lane], skey)
                plsc.store_scatter(valref, [base + lane], sval)
                return 0
            lax.fori_loop(0, nb, body, 0)

        def cross_sweep(kk, j):
            j16 = j // 16
            k16 = kk // 16
            def body(b, _):
                @pl.when((b & j16) == 0)
                def _():
                    base_a = b * 16
                    base_b = (b + j16) * 16
                    ka = plsc.load_gather(keyref, [base_a + lane])
                    va = plsc.load_gather(valref, [base_a + lane])
                    kb = plsc.load_gather(keyref, [base_b + lane])
                    vb = plsc.load_gather(valref, [base_b + lane])
                    desc = (b & k16) == 0
                    swap = jnp.where(desc, ka < kb, ka > kb)
                    plsc.store_scatter(keyref, [base_a + lane],
                                       jnp.where(swap, kb, ka))
                    plsc.store_scatter(keyref, [base_b + lane],
                                       jnp.where(swap, ka, kb))
                    plsc.store_scatter(valref, [base_a + lane],
                                       jnp.where(swap, vb, va))
                    plsc.store_scatter(valref, [base_b + lane],
                                       jnp.where(swap, va, vb))
                return 0
            lax.fori_loop(0, nb, body, 0)

        vsort_sweep(16)
        kk = 32
        while kk <= n:
            j = kk // 2
            while j >= 16:
                cross_sweep(kk, j)
                j //= 2
            vsort_sweep(kk)
            kk *= 2

    @pl.when(s < _NCL)
    def _phase1():
        row = c * _NCL + s
        pltpu.sync_copy(scores_hbm.at[pl.ds(row * _PADN, _PADN)], sv)

        def load_chunk(j):
            v = plsc.load_gather(sv, [j * 16 + lane])
            m = v > _CONF_THRESH
            k = plsc.bitcast(v, jnp.int32)
            return v, m, k

        hist_zero()
        def bucket_a(j):
            _, m, k = load_chunk(j)
            return lax.shift_right_logical(k, 23), m
        hist_pass(bucket_a)
        b1, ca1, found1 = hist_select(jnp.int32(_TOPK))

        hist_zero()
        def bucket_b(j):
            _, m, k = load_chunk(j)
            m2 = jnp.logical_and(m, lax.shift_right_logical(k, 23) == b1)
            return jnp.bitwise_and(lax.shift_right_logical(k, 15), 255), m2
        hist_pass(bucket_b)
        b2, _, _ = hist_select(_TOPK - ca1)
        lo = jnp.where(found1,
                       jnp.bitwise_or(lax.shift_left(b1, 23),
                                      lax.shift_left(b2, 15)),
                       jnp.int32(0))

        def initcb(j, _):
            plsc.store_scatter(cb_key, [j * 16 + lane], zero16)
            plsc.store_scatter(cb_idx, [j * 16 + lane],
                               jnp.full((16,), _ZROW, jnp.int32))
            return 0
        lax.fori_loop(0, _CBUF // 16, initcb, 0)

        def coll(j, off):
            idxv = j * 16 + lane
            v = plsc.load_gather(sv, [idxv])
            m = v > _CONF_THRESH
            k = plsc.bitcast(v, jnp.int32)
            cm = jnp.logical_and(m, k >= lo)
            cmi = cm.astype(jnp.int32)
            pos = off + plsc.cumsum(cmi) - 1
            guard = jnp.logical_and(cm, pos < _CCAP)
            plsc.store_scatter(cb_key, [pos], k, mask=guard)
            plsc.store_scatter(cb_idx, [pos], idxv, mask=guard)
            return jnp.minimum(off + jnp.sum(cmi), _CCAP)
        lax.fori_loop(0, _NVEC, coll, jnp.int32(0))

        bitonic(cb_key, cb_idx, _CBUF)

        def initidx(j, _):
            iv = plsc.load_gather(cb_idx, [j * 16 + lane])
            plsc.store_scatter(outp, [j * 16 + lane], iv + c * _PADN)
            return 0
        lax.fori_loop(0, _CAND // 16, initidx, 0)
        pltpu.async_copy(payload_hbm.at[outp], cand_rows, sem).wait()

        def getcol(j, _):
            r = j * 16 + lane
            x1 = plsc.load_gather(cand_rows, [r, jnp.full((16,), 1, jnp.int32)])
            y1 = plsc.load_gather(cand_rows, [r, jnp.full((16,), 2, jnp.int32)])
            x2 = plsc.load_gather(cand_rows, [r, jnp.full((16,), 3, jnp.int32)])
            y2 = plsc.load_gather(cand_rows, [r, jnp.full((16,), 4, jnp.int32)])
            plsc.store_scatter(bx, [0 * _CAND + r], x1)
            plsc.store_scatter(bx, [1 * _CAND + r], y1)
            plsc.store_scatter(bx, [2 * _CAND + r], x2)
            plsc.store_scatter(bx, [3 * _CAND + r], y2)
            plsc.store_scatter(bx, [4 * _CAND + r], (x2 - x1) * (y2 - y1))
            plsc.store_scatter(keep, [r], zero16)
            return 0
        lax.fori_loop(0, _CAND // 16, getcol, 0)

        def nms_step(i, _):
            i16 = jnp.full((16,), i, jnp.int32)
            xx1 = plsc.load_gather(bx, [0 * _CAND + i16])
            yy1 = plsc.load_gather(bx, [1 * _CAND + i16])
            xx2 = plsc.load_gather(bx, [2 * _CAND + i16])
            yy2 = plsc.load_gather(bx, [3 * _CAND + i16])
            aar = plsc.load_gather(bx, [4 * _CAND + i16])
            key_i = plsc.load_gather(cb_key, [i16])[0]
            def chunk(j, acc):
                r = j * 16 + lane
                x1 = plsc.load_gather(bx, [0 * _CAND + r])
                y1 = plsc.load_gather(bx, [1 * _CAND + r])
                x2 = plsc.load_gather(bx, [2 * _CAND + r])
                y2 = plsc.load_gather(bx, [3 * _CAND + r])
                ar = plsc.load_gather(bx, [4 * _CAND + r])
                kp = plsc.load_gather(keep, [r])
                iw = jnp.maximum(jnp.minimum(x2, xx2) - jnp.maximum(x1, xx1),
                                 0.0)
                ih = jnp.maximum(jnp.minimum(y2, yy2) - jnp.maximum(y1, yy1),
                                 0.0)
                inter = iw * ih
                iou = inter / (ar + aar - inter)
                hit = jnp.logical_and(kp > 0, iou > _NMS_THRESH)
                return jnp.logical_or(acc, hit)
            accv = lax.fori_loop(0, _CAND // 16, chunk,
                                 jnp.zeros((16,), jnp.bool_))
            sup = jnp.any(accv)
            kv = jnp.logical_and(key_i > _BITS001,
                                 jnp.logical_not(sup)).astype(jnp.int32)
            plsc.store_scatter(keep, [i16], jnp.full((16,), kv, jnp.int32),
                               mask=lane == 0)
            return 0
        lax.fori_loop(0, _TOPK, nms_step, 0)

        pltpu.sync_copy(cb_idx.at[pl.ds(0, _CAND)], shared.at[s, 0])
        pltpu.sync_copy(cb_key.at[pl.ds(0, _CAND)], shared.at[s, 1])
        pltpu.sync_copy(keep, shared.at[s, 2])

    plsc.subcore_barrier()

    @pl.when(s == _NCL)
    def _phase2():
        def initm(j, _):
            idxv = j * 16 + lane
            plsc.store_scatter(m_key, [idxv], zero16)
            plsc.store_scatter(m_prior, [idxv],
                               jnp.full((16,), _ZROW, jnp.int32))
            plsc.store_scatter(m_keep, [idxv], zero16)
            return 0
        lax.fori_loop(0, _MRG // 16, initm, 0)
        for u in range(_NCL):
            pltpu.sync_copy(shared.at[u, 0], m_prior.at[pl.ds(u * _CAND, _CAND)])
            pltpu.sync_copy(shared.at[u, 1], m_key.at[pl.ds(u * _CAND, _CAND)])
            pltpu.sync_copy(shared.at[u, 2], m_keep.at[pl.ds(u * _CAND, _CAND)])
        def zs(j, _):
            plsc.store_scatter(seen, [j * 16 + lane], zero16)
            return 0
        lax.fori_loop(0, _NVEC, zs, 0)
        for u in range(_NCL):
            def dd(j, _):
                sl = u * _CAND + j * 16 + lane
                p = plsc.load_gather(m_prior, [sl])
                kp = plsc.load_gather(m_keep, [sl]) > 0
                dup = plsc.load_gather(seen, [p]) > 0
                k = plsc.load_gather(m_key, [sl])
                newk = jnp.where(
                    jnp.logical_and(kp, jnp.logical_not(dup)), k, 0)
                plsc.store_scatter(m_key, [sl], newk)
                plsc.store_scatter(seen, [p], ones16, mask=kp)
                return 0
            lax.fori_loop(0, _CAND // 16, dd, 0)
        bitonic(m_key, m_prior, _MRG)
        def mkout(j, _):
            r = j * 16 + lane
            k = plsc.load_gather(m_key, [r])
            p = plsc.load_gather(m_prior, [r])
            o = jnp.where(k > _BITS001, p, _ZROW) + c * _PADN
            plsc.store_scatter(outp, [r], o)
            return 0
        lax.fori_loop(0, _CAND // 16, mkout, 0)
        pltpu.async_copy(payload_hbm.at[outp], cand_rows, sem).wait()
        pltpu.sync_copy(cand_rows, out_hbm.at[c])


def _sc_select(scores, payload):
    mesh = plsc.VectorSubcoreMesh(core_axis_name="c", subcore_axis_name="s",
                                  num_cores=_B, num_subcores=16)
    f = pl.kernel(
        _sc_select_body,
        out_type=jax.ShapeDtypeStruct((_B, _CAND, _PCOLS), jnp.float32),
        mesh=mesh,
        compiler_params=pltpu.CompilerParams(needs_layout_passes=False),
        scratch_types=[
            pltpu.VMEM((_PADN,), jnp.float32),
            pltpu.VMEM((_CBUF,), jnp.int32),
            pltpu.VMEM((_CBUF,), jnp.int32),
            pltpu.VMEM((4096,), jnp.int32),
            pltpu.VMEM((_CAND, _PCOLS), jnp.float32),
            pltpu.VMEM((5 * _CAND,), jnp.float32),
            pltpu.VMEM((_CAND,), jnp.int32),
            pltpu.VMEM((_MRG,), jnp.int32),
            pltpu.VMEM((_MRG,), jnp.int32),
            pltpu.VMEM((_MRG,), jnp.int32),
            pltpu.VMEM((_PADN,), jnp.int32),
            pltpu.VMEM((_CAND,), jnp.int32),
            pltpu.VMEM_SHARED((16, 3, _CAND), jnp.int32),
            pltpu.SemaphoreType.DMA,
        ],
    )
    return f(scores.reshape(-1), payload)


def _select_jax(scores, payload):
    outs = []
    for i in range(_B):
        keys_all, prior_all, kept_all = [], [], []
        for c in range(_NCL):
            s = scores[i * _NCL + c]
            masked = jnp.where(s > _CONF_THRESH, s, -jnp.inf)
            _, cand = lax.top_k(masked, _TOPK)
            sv = s[cand]
            valid = sv > _CONF_THRESH
            rows = payload[i * _PADN + cand]
            x1, y1, x2, y2 = rows[:, 1], rows[:, 2], rows[:, 3], rows[:, 4]
            area = (x2 - x1) * (y2 - y1)
            ix1 = jnp.maximum(x1[:, None], x1[None, :])
            iy1 = jnp.maximum(y1[:, None], y1[None, :])
            ix2 = jnp.minimum(x2[:, None], x2[None, :])
            iy2 = jnp.minimum(y2[:, None], y2[None, :])
            iw = jnp.clip(ix2 - ix1, 0.0, None)
            ih = jnp.clip(iy2 - iy1, 0.0, None)
            inter = iw * ih
            iou = inter / (area[:, None] + area[None, :] - inter)
            keep = jnp.zeros((_TOPK,), bool)
            def nms_step(k, keep):
                sup = jnp.any(keep & (iou[:, k] > _NMS_THRESH))
                return keep.at[k].set(valid[k] & jnp.logical_not(sup))
            keep = lax.fori_loop(0, _TOPK, nms_step, keep)
            keys_all.append(jnp.where(keep, sv, 0.0))
            prior_all.append(cand)
            kept_all.append(keep)
        keys = jnp.concatenate(keys_all)
        prior = jnp.concatenate(prior_all)
        kept = jnp.concatenate(kept_all)
        M = keys.shape[0]
        same = prior[:, None] == prior[None, :]
        earlier = jnp.arange(M)[None, :] < jnp.arange(M)[:, None]
        dup = jnp.any(same & earlier & kept[None, :], axis=1)
        final = kept & jnp.logical_not(dup)
        key_bits = jnp.where(final, keys, 0.0)
        order = jnp.argsort(-key_bits)[:_TOPK]
        sel_prior = jnp.where(key_bits[order] > _CONF_THRESH,
                              prior[order], _ZROW)
        outs.append(payload[i * _PADN + sel_prior])
    return jnp.stack(outs)


def kernel(loc_data, conf_data, prior_data, targets):
    scores, payload = _tc_prep(loc_data, conf_data, prior_data, targets)
    result = _sc_select(scores, payload)[:, :_TOPK, :]
    rois = result[..., 0:5]
    loc = result[..., 5:9]
    cls = result[..., 9:9 + _C]
    loc_truth = result[..., 9 + _C:13 + _C]
    conf_truth = result[..., 13 + _C:14 + _C]
    return rois, loc, cls, loc_truth, conf_truth


# speedup vs baseline: 36.2622x; 1.2401x over previous
import functools
import struct

import jax
import jax.numpy as jnp
from jax import lax
from jax.experimental import pallas as pl
from jax.experimental.pallas import tpu as pltpu
from jax.experimental.pallas import tpu_sc as plsc

_C = 6
_NCL = _C - 1
_B = 2
_NP = 5000
_PADN = 5120
_TOPK = 100
_CAND = 112
_OVERLAP_THRESH = 0.5
_CONF_THRESH = 0.01
_NMS_THRESH = 0.45
_VAR0 = 0.1
_VAR1 = 0.2
_NOBJ = 8
_COLS = 24
_PCOLS = 128
_ZROW = _NP
_BITS001 = struct.unpack("<i", struct.pack("<f", _CONF_THRESH))[0]


def _tc_prep_body(locT_ref, confT_ref, priorsT_ref, targets_ref,
                  scores_ref, payload_ref):
    cx = priorsT_ref[0:1, :]
    cy = priorsT_ref[1:2, :]
    pw = priorsT_ref[2:3, :]
    ph = priorsT_ref[3:4, :]
    col = lax.broadcasted_iota(jnp.int32, (1, _PADN), 1)
    real = col < _NP
    realf = real.astype(jnp.float32)
    px1 = cx - pw * 0.5
    py1 = cy - ph * 0.5
    px2 = cx + pw * 0.5
    py2 = cy + ph * 0.5
    p_area = (px2 - px1) * (py2 - py1)

    for i in range(_B):
        tx1 = targets_ref[i, :, 0:1]
        ty1 = targets_ref[i, :, 1:2]
        tx2 = targets_ref[i, :, 2:3]
        ty2 = targets_ref[i, :, 3:4]
        tlab = targets_ref[i, :, 4:5]
        t_area = (tx2 - tx1) * (ty2 - ty1)
        ix1 = jnp.maximum(tx1, px1)
        iy1 = jnp.maximum(ty1, py1)
        ix2 = jnp.minimum(tx2, px2)
        iy2 = jnp.minimum(ty2, py2)
        iw = jnp.clip(ix2 - ix1, 0.0, None)
        ih = jnp.clip(iy2 - iy1, 0.0, None)
        inter = iw * ih
        ov = inter / (t_area + p_area - inter)
        ov = jnp.where(real, ov, -1.0)
        bigcol = jnp.where(ov == jnp.max(ov, axis=1, keepdims=True),
                           jnp.broadcast_to(col, ov.shape), _PADN)
        best_prior_idx = jnp.min(bigcol, axis=1, keepdims=True)
        trow = lax.broadcasted_iota(jnp.int32, (_NOBJ, 1), 0)
        ovmax0 = jnp.max(ov, axis=0, keepdims=True)
        bigrow = jnp.where(ov == ovmax0,
                           jnp.broadcast_to(trow, ov.shape), _NOBJ)
        best_truth_idx = jnp.min(bigrow, axis=0, keepdims=True)
        best_truth_overlap = ovmax0
        for t in range(_NOBJ):
            hit = col == best_prior_idx[t, 0]
            best_truth_overlap = jnp.where(hit, 2.0, best_truth_overlap)
            best_truth_idx = jnp.where(hit, t, best_truth_idx)
        mx1 = jnp.zeros((1, _PADN), jnp.float32)
        my1 = jnp.zeros((1, _PADN), jnp.float32)
        mx2 = jnp.zeros((1, _PADN), jnp.float32)
        my2 = jnp.zeros((1, _PADN), jnp.float32)
        mlab = jnp.zeros((1, _PADN), jnp.float32)
        for t in range(_NOBJ):
            sel = best_truth_idx == t
            mx1 = jnp.where(sel, tx1[t, 0], mx1)
            my1 = jnp.where(sel, ty1[t, 0], my1)
            mx2 = jnp.where(sel, tx2[t, 0], mx2)
            my2 = jnp.where(sel, ty2[t, 0], my2)
            mlab = jnp.where(sel, tlab[t, 0], mlab)
        conf_t = jnp.where(best_truth_overlap < _OVERLAP_THRESH, 0.0,
                           mlab + 1.0)
        g_cx = ((mx1 + mx2) * 0.5 - cx) / (_VAR0 * pw)
        g_cy = ((my1 + my2) * 0.5 - cy) / (_VAR0 * ph)
        safe_w = jnp.where(real, (mx2 - mx1) / pw, 1.0)
        safe_h = jnp.where(real, (my2 - my1) / ph, 1.0)
        g_w = jnp.log(safe_w) / _VAR1
        g_h = jnp.log(safe_h) / _VAR1

        l0 = locT_ref[i, 0:1, :]
        l1 = locT_ref[i, 1:2, :]
        l2 = locT_ref[i, 2:3, :]
        l3 = locT_ref[i, 3:4, :]
        dcx = cx + l0 * _VAR0 * pw
        dcy = cy + l1 * _VAR0 * ph
        dw = pw * jnp.exp(l2 * _VAR1)
        dh = ph * jnp.exp(l3 * _VAR1)
        dx1 = dcx - dw * 0.5
        dy1 = dcy - dh * 0.5
        dx2 = dx1 + dw
        dy2 = dy1 + dh

        for c in range(1, _C):
            srow = confT_ref[i, c:c + 1, :] * realf
            scores_ref[i * _NCL + (c - 1), :] = srow[0]

        cols = [jnp.full((1, _PADN), float(i), jnp.float32),
                dx1, dy1, dx2, dy2,
                l0, l1, l2, l3]
        for c in range(_C):
            cols.append(confT_ref[i, c:c + 1, :])
        cols += [g_cx, g_cy, g_w, g_h, conf_t]
        while len(cols) < _COLS:
            cols.append(jnp.zeros((1, _PADN), jnp.float32))
        tab = jnp.concatenate([v * realf for v in cols], axis=0)
        payload_ref[i, :, 0:_COLS] = jnp.transpose(tab, (1, 0))
        payload_ref[i, :, _COLS:_PCOLS] = jnp.zeros(
            (_PADN, _PCOLS - _COLS), jnp.float32)


def _tc_prep(loc_data, conf_data, prior_data, targets):
    locT = jnp.pad(jnp.transpose(loc_data, (0, 2, 1)),
                   ((0, 0), (0, 0), (0, _PADN - _NP)))
    conf3 = conf_data.reshape(_B, _NP, _C)
    confT = jnp.pad(jnp.transpose(conf3, (0, 2, 1)),
                    ((0, 0), (0, 0), (0, _PADN - _NP)))
    priorsT = jnp.pad(prior_data.T, ((0, 0), (0, _PADN - _NP)),
                      constant_values=1.0)
    padmask = jnp.arange(_PADN)[None, :] >= _NP
    priorsT = jnp.where(padmask & (jnp.arange(4)[:, None] < 2), 100.0, priorsT)
    scores, payload = pl.pallas_call(
        _tc_prep_body,
        out_shape=[
            jax.ShapeDtypeStruct((_B * _NCL, _PADN), jnp.float32),
            jax.ShapeDtypeStruct((_B, _PADN, _PCOLS), jnp.float32),
        ],
    )(locT, confT, priorsT, targets)
    return scores, payload.reshape(_B * _PADN, _PCOLS)


_CBUF = 512
_CCAP = _CBUF - 16
_MRG = 1024
_NVEC = _PADN // 16


def _sc_select_body(scores_hbm, payload_hbm, out_hbm,
                    sv, cb_key, cb_idx, hist, cand_rows, bx, keep,
                    m_key, m_prior, m_keep, seen, outp, shared, sem):
    c = lax.axis_index("c")
    s = lax.axis_index("s")
    lane = lax.iota(jnp.int32, 16)
    ones16 = jnp.ones((16,), jnp.int32)
    zero16 = jnp.zeros((16,), jnp.int32)

    def hist_zero():
        def z(j, _):
            plsc.store_scatter(hist, [j * 16 + lane], zero16)
            return 0
        lax.fori_loop(0, 256, z, 0)

    def hist_pass(bucket_fn):
        def body(j, _):
            b, m = bucket_fn(j)
            plsc.addupdate_scatter(hist, [lane * 256 + b], ones16, mask=m)
            return 0
        lax.fori_loop(0, _NVEC, body, 0)

    def hist_select(rank):
        run = jnp.int32(0)
        B = jnp.int32(0)
        ca = jnp.int32(0)
        found = jnp.bool_(False)
        for g in range(15, -1, -1):
            acc = zero16
            for l in range(16):
                acc = acc + hist[pl.ds(l * 256 + g * 16, 16)]
            sfx = lax.rev(plsc.cumsum(lax.rev(acc, (0,))), (0,)) + run
            mask = sfx >= rank
            cnt = plsc.all_reduce_population_count(mask)[0]
            cav = jnp.sum(jnp.where(lane == cnt, sfx, 0))
            ca_g = jnp.where(cnt == 16, run, cav)
            hit = jnp.logical_and(jnp.logical_not(found), cnt > 0)
            B = jnp.where(hit, g * 16 + cnt - 1, B)
            ca = jnp.where(hit, ca_g, ca)
            found = jnp.logical_or(found, cnt > 0)
            run = run + jnp.sum(acc)
        return B, ca, found

    def bitonic(keyref, valref, n):
        nb = n // 16

        def vsort_sweep(kk):
            k16 = kk // 16
            def body(b, _):
                base = b * 16
                key = plsc.load_gather(keyref, [base + lane])
                val = plsc.load_gather(valref, [base + lane])
                desc = (b & k16) == 0
                tkey = jnp.where(desc, key, -1 - key)
                skey, sval = plsc.sort_key_val(tkey, val, descending=True)
                skey = jnp.where(desc, skey, -1 - skey)
                plsc.store_scatter(keyref, [base + lane], skey)
                plsc.store_scatter(valref, [base + lane], sval)
                return 0
            lax.fori_loop(0, nb, body, 0)

        def cross_sweep(kk, j):
            j16 = j // 16
            k16 = kk // 16
            def body(b, _):
                @pl.when((b & j16) == 0)
                def _():
                    base_a = b * 16
                    base_b = (b + j16) * 16
                    ka = plsc.load_gather(keyref, [base_a + lane])
                    va = plsc.load_gather(valref, [base_a + lane])
                    kb = plsc.load_gather(keyref, [base_b + lane])
                    vb = plsc.load_gather(valref, [base_b + lane])
                    desc = (b & k16) == 0
                    swap = jnp.where(desc, ka < kb, ka > kb)
                    plsc.store_scatter(keyref, [base_a + lane],
                                       jnp.where(swap, kb, ka))
                    plsc.store_scatter(keyref, [base_b + lane],
                                       jnp.where(swap, ka, kb))
                    plsc.store_scatter(valref, [base_a + lane],
                                       jnp.where(swap, vb, va))
                    plsc.store_scatter(valref, [base_b + lane],
                                       jnp.where(swap, va, vb))
                return 0
            lax.fori_loop(0, nb, body, 0)

        vsort_sweep(16)
        kk = 32
        while kk <= n:
            j = kk // 2
            while j >= 16:
                cross_sweep(kk, j)
                j //= 2
            vsort_sweep(kk)
            kk *= 2

    @pl.when(s < _NCL)
    def _phase1():
        row = c * _NCL + s
        pltpu.sync_copy(scores_hbm.at[pl.ds(row * _PADN, _PADN)], sv)

        def load_chunk(j):
            v = plsc.load_gather(sv, [j * 16 + lane])
            m = v > _CONF_THRESH
            k = plsc.bitcast(v, jnp.int32)
            return v, m, k

        hist_zero()
        def bucket_a(j):
            _, m, k = load_chunk(j)
            return lax.shift_right_logical(k, 23), m
        hist_pass(bucket_a)
        b1, ca1, found1 = hist_select(jnp.int32(_TOPK))

        hist_zero()
        def bucket_b(j):
            _, m, k = load_chunk(j)
            m2 = jnp.logical_and(m, lax.shift_right_logical(k, 23) == b1)
            return jnp.bitwise_and(lax.shift_right_logical(k, 15), 255), m2
        hist_pass(bucket_b)
        b2, _, _ = hist_select(_TOPK - ca1)
        lo = jnp.where(found1,
                       jnp.bitwise_or(lax.shift_left(b1, 23),
                                      lax.shift_left(b2, 15)),
                       jnp.int32(0))

        def initcb(j, _):
            plsc.store_scatter(cb_key, [j * 16 + lane], zero16)
            plsc.store_scatter(cb_idx, [j * 16 + lane],
                               jnp.full((16,), _ZROW, jnp.int32))
            return 0
        lax.fori_loop(0, _CBUF // 16, initcb, 0)

        def coll(j, off):
            idxv = j * 16 + lane
            v = plsc.load_gather(sv, [idxv])
            m = v > _CONF_THRESH
            k = plsc.bitcast(v, jnp.int32)
            cm = jnp.logical_and(m, k >= lo)
            cmi = cm.astype(jnp.int32)
            pos = off + plsc.cumsum(cmi) - 1
            guard = jnp.logical_and(cm, pos < _CCAP)
            plsc.store_scatter(cb_key, [pos], k, mask=guard)
            plsc.store_scatter(cb_idx, [pos], idxv, mask=guard)
            return jnp.minimum(off + jnp.sum(cmi), _CCAP)
        off = lax.fori_loop(0, _NVEC, coll, jnp.int32(0))

        @pl.when(off <= 128)
        def _small():
            bitonic(cb_key, cb_idx, 128)

        @pl.when(off > 128)
        def _big():
            bitonic(cb_key, cb_idx, _CBUF)

        def initidx(j, _):
            iv = plsc.load_gather(cb_idx, [j * 16 + lane])
            plsc.store_scatter(outp, [j * 16 + lane], iv + c * _PADN)
            return 0
        lax.fori_loop(0, _CAND // 16, initidx, 0)
        pltpu.async_copy(payload_hbm.at[outp], cand_rows, sem).wait()

        def getcol(j, _):
            r = j * 16 + lane
            x1 = plsc.load_gather(cand_rows, [r, jnp.full((16,), 1, jnp.int32)])
            y1 = plsc.load_gather(cand_rows, [r, jnp.full((16,), 2, jnp.int32)])
            x2 = plsc.load_gather(cand_rows, [r, jnp.full((16,), 3, jnp.int32)])
            y2 = plsc.load_gather(cand_rows, [r, jnp.full((16,), 4, jnp.int32)])
            plsc.store_scatter(bx, [0 * _CAND + r], x1)
            plsc.store_scatter(bx, [1 * _CAND + r], y1)
            plsc.store_scatter(bx, [2 * _CAND + r], x2)
            plsc.store_scatter(bx, [3 * _CAND + r], y2)
            plsc.store_scatter(bx, [4 * _CAND + r], (x2 - x1) * (y2 - y1))
            plsc.store_scatter(keep, [r], zero16)
            return 0
        lax.fori_loop(0, _CAND // 16, getcol, 0)

        def nms_step(i, _):
            i16 = jnp.full((16,), i, jnp.int32)
            xx1 = plsc.load_gather(bx, [0 * _CAND + i16])
            yy1 = plsc.load_gather(bx, [1 * _CAND + i16])
            xx2 = plsc.load_gather(bx, [2 * _CAND + i16])
            yy2 = plsc.load_gather(bx, [3 * _CAND + i16])
            aar = plsc.load_gather(bx, [4 * _CAND + i16])
            key_i = plsc.load_gather(cb_key, [i16])[0]
            def chunk(j, acc):
                r = j * 16 + lane
                x1 = plsc.load_gather(bx, [0 * _CAND + r])
                y1 = plsc.load_gather(bx, [1 * _CAND + r])
                x2 = plsc.load_gather(bx, [2 * _CAND + r])
                y2 = plsc.load_gather(bx, [3 * _CAND + r])
                ar = plsc.load_gather(bx, [4 * _CAND + r])
                kp = plsc.load_gather(keep, [r])
                iw = jnp.maximum(jnp.minimum(x2, xx2) - jnp.maximum(x1, xx1),
                                 0.0)
                ih = jnp.maximum(jnp.minimum(y2, yy2) - jnp.maximum(y1, yy1),
                                 0.0)
                inter = iw * ih
                iou = inter / (ar + aar - inter)
                hit = jnp.logical_and(kp > 0, iou > _NMS_THRESH)
                return jnp.logical_or(acc, hit)
            accv = lax.fori_loop(0, lax.shift_right_logical(i, 4) + 1, chunk,
                                 jnp.zeros((16,), jnp.bool_))
            sup = jnp.any(accv)
            kv = jnp.logical_and(key_i > _BITS001,
                                 jnp.logical_not(sup)).astype(jnp.int32)
            plsc.store_scatter(keep, [i16], jnp.full((16,), kv, jnp.int32),
                               mask=lane == 0)
            return 0
        lax.fori_loop(0, _TOPK, nms_step, 0)

        pltpu.sync_copy(cb_idx.at[pl.ds(0, _CAND)], shared.at[s, 0])
        pltpu.sync_copy(cb_key.at[pl.ds(0, _CAND)], shared.at[s, 1])
        pltpu.sync_copy(keep, shared.at[s, 2])

    plsc.subcore_barrier()

    @pl.when(s == _NCL)
    def _phase2():
        def initm(j, _):
            idxv = j * 16 + lane
            plsc.store_scatter(m_key, [idxv], zero16)
            plsc.store_scatter(m_prior, [idxv],
                               jnp.full((16,), _ZROW, jnp.int32))
            plsc.store_scatter(m_keep, [idxv], zero16)
            return 0
        lax.fori_loop(0, _MRG // 16, initm, 0)
        for u in range(_NCL):
            pltpu.sync_copy(shared.at[u, 0], m_prior.at[pl.ds(u * _CAND, _CAND)])
            pltpu.sync_copy(shared.at[u, 1], m_key.at[pl.ds(u * _CAND, _CAND)])
            pltpu.sync_copy(shared.at[u, 2], m_keep.at[pl.ds(u * _CAND, _CAND)])
        def zs(j, _):
            plsc.store_scatter(seen, [j * 16 + lane], zero16)
            return 0
        lax.fori_loop(0, _NVEC, zs, 0)
        for u in range(_NCL):
            def dd(j, _):
                sl = u * _CAND + j * 16 + lane
                p = plsc.load_gather(m_prior, [sl])
                kp = plsc.load_gather(m_keep, [sl]) > 0
                dup = plsc.load_gather(seen, [p]) > 0
                k = plsc.load_gather(m_key, [sl])
                newk = jnp.where(
                    jnp.logical_and(kp, jnp.logical_not(dup)), k, 0)
                plsc.store_scatter(m_key, [sl], newk)
                plsc.store_scatter(seen, [p], ones16, mask=kp)
                return 0
            lax.fori_loop(0, _CAND // 16, dd, 0)
        def initcb2(j, _):
            plsc.store_scatter(cb_key, [j * 16 + lane], zero16)
            plsc.store_scatter(cb_idx, [j * 16 + lane],
                               jnp.full((16,), _ZROW, jnp.int32))
            return 0
        lax.fori_loop(0, _CBUF // 16, initcb2, 0)

        def compress(j, off):
            sl = j * 16 + lane
            k = plsc.load_gather(m_key, [sl])
            p = plsc.load_gather(m_prior, [sl])
            m = k > _BITS001
            mi = m.astype(jnp.int32)
            pos = off + plsc.cumsum(mi) - 1
            plsc.store_scatter(cb_key, [pos], k, mask=m)
            plsc.store_scatter(cb_idx, [pos], p, mask=m)
            return off + jnp.sum(mi)
        lax.fori_loop(0, (_NCL * _CAND) // 16, compress, jnp.int32(0))

        bitonic(cb_key, cb_idx, _CBUF)
        def mkout(j, _):
            r = j * 16 + lane
            k = plsc.load_gather(cb_key, [r])
            p = plsc.load_gather(cb_idx, [r])
            o = jnp.where(k > _BITS001, p, _ZROW) + c * _PADN
            plsc.store_scatter(outp, [r], o)
            return 0
        lax.fori_loop(0, _CAND // 16, mkout, 0)
        pltpu.async_copy(payload_hbm.at[outp], cand_rows, sem).wait()
        pltpu.sync_copy(cand_rows, out_hbm.at[c])


def _sc_select(scores, payload):
    mesh = plsc.VectorSubcoreMesh(core_axis_name="c", subcore_axis_name="s",
                                  num_cores=_B, num_subcores=16)
    f = pl.kernel(
        _sc_select_body,
        out_type=jax.ShapeDtypeStruct((_B, _CAND, _PCOLS), jnp.float32),
        mesh=mesh,
        compiler_params=pltpu.CompilerParams(needs_layout_passes=False),
        scratch_types=[
            pltpu.VMEM((_PADN,), jnp.float32),
            pltpu.VMEM((_CBUF,), jnp.int32),
            pltpu.VMEM((_CBUF,), jnp.int32),
            pltpu.VMEM((4096,), jnp.int32),
            pltpu.VMEM((_CAND, _PCOLS), jnp.float32),
            pltpu.VMEM((5 * _CAND,), jnp.float32),
            pltpu.VMEM((_CAND,), jnp.int32),
            pltpu.VMEM((_MRG,), jnp.int32),
            pltpu.VMEM((_MRG,), jnp.int32),
            pltpu.VMEM((_MRG,), jnp.int32),
            pltpu.VMEM((_PADN,), jnp.int32),
            pltpu.VMEM((_CAND,), jnp.int32),
            pltpu.VMEM_SHARED((16, 3, _CAND), jnp.int32),
            pltpu.SemaphoreType.DMA,
        ],
    )
    return f(scores.reshape(-1), payload)


def _select_jax(scores, payload):
    outs = []
    for i in range(_B):
        keys_all, prior_all, kept_all = [], [], []
        for c in range(_NCL):
            s = scores[i * _NCL + c]
            masked = jnp.where(s > _CONF_THRESH, s, -jnp.inf)
            _, cand = lax.top_k(masked, _TOPK)
            sv = s[cand]
            valid = sv > _CONF_THRESH
            rows = payload[i * _PADN + cand]
            x1, y1, x2, y2 = rows[:, 1], rows[:, 2], rows[:, 3], rows[:, 4]
            area = (x2 - x1) * (y2 - y1)
            ix1 = jnp.maximum(x1[:, None], x1[None, :])
            iy1 = jnp.maximum(y1[:, None], y1[None, :])
            ix2 = jnp.minimum(x2[:, None], x2[None, :])
            iy2 = jnp.minimum(y2[:, None], y2[None, :])
            iw = jnp.clip(ix2 - ix1, 0.0, None)
            ih = jnp.clip(iy2 - iy1, 0.0, None)
            inter = iw * ih
            iou = inter / (area[:, None] + area[None, :] - inter)
            keep = jnp.zeros((_TOPK,), bool)
            def nms_step(k, keep):
                sup = jnp.any(keep & (iou[:, k] > _NMS_THRESH))
                return keep.at[k].set(valid[k] & jnp.logical_not(sup))
            keep = lax.fori_loop(0, _TOPK, nms_step, keep)
            keys_all.append(jnp.where(keep, sv, 0.0))
            prior_all.append(cand)
            kept_all.append(keep)
        keys = jnp.concatenate(keys_all)
        prior = jnp.concatenate(prior_all)
        kept = jnp.concatenate(kept_all)
        M = keys.shape[0]
        same = prior[:, None] == prior[None, :]
        earlier = jnp.arange(M)[None, :] < jnp.arange(M)[:, None]
        dup = jnp.any(same & earlier & kept[None, :], axis=1)
        final = kept & jnp.logical_not(dup)
        key_bits = jnp.where(final, keys, 0.0)
        order = jnp.argsort(-key_bits)[:_TOPK]
        sel_prior = jnp.where(key_bits[order] > _CONF_THRESH,
                              prior[order], _ZROW)
        outs.append(payload[i * _PADN + sel_prior])
    return jnp.stack(outs)


def kernel(loc_data, conf_data, prior_data, targets):
    scores, payload = _tc_prep(loc_data, conf_data, prior_data, targets)
    result = _sc_select(scores, payload)[:, :_TOPK, :]
    rois = result[..., 0:5]
    loc = result[..., 5:9]
    cls = result[..., 9:9 + _C]
    loc_truth = result[..., 9 + _C:13 + _C]
    conf_truth = result[..., 13 + _C:14 + _C]
    return rois, loc, cls, loc_truth, conf_truth


# speedup vs baseline: 38.2441x; 1.0547x over previous
import functools
import struct

import jax
import jax.numpy as jnp
from jax import lax
from jax.experimental import pallas as pl
from jax.experimental.pallas import tpu as pltpu
from jax.experimental.pallas import tpu_sc as plsc

_C = 6
_NCL = _C - 1
_B = 2
_NP = 5000
_PADN = 5120
_TOPK = 100
_CAND = 112
_OVERLAP_THRESH = 0.5
_CONF_THRESH = 0.01
_NMS_THRESH = 0.45
_VAR0 = 0.1
_VAR1 = 0.2
_NOBJ = 8
_COLS = 24
_PCOLS = 128
_ZROW = _NP
_BITS001 = struct.unpack("<i", struct.pack("<f", _CONF_THRESH))[0]


def _tc_prep_body(locT_ref, confT_ref, priorsT_ref, targets_ref,
                  scores_ref, payload_ref):
    cx = priorsT_ref[0:1, :]
    cy = priorsT_ref[1:2, :]
    pw = priorsT_ref[2:3, :]
    ph = priorsT_ref[3:4, :]
    col = lax.broadcasted_iota(jnp.int32, (1, _PADN), 1)
    real = col < _NP
    realf = real.astype(jnp.float32)
    px1 = cx - pw * 0.5
    py1 = cy - ph * 0.5
    px2 = cx + pw * 0.5
    py2 = cy + ph * 0.5
    p_area = (px2 - px1) * (py2 - py1)

    for i in range(_B):
        tx1 = targets_ref[i, :, 0:1]
        ty1 = targets_ref[i, :, 1:2]
        tx2 = targets_ref[i, :, 2:3]
        ty2 = targets_ref[i, :, 3:4]
        tlab = targets_ref[i, :, 4:5]
        t_area = (tx2 - tx1) * (ty2 - ty1)
        ix1 = jnp.maximum(tx1, px1)
        iy1 = jnp.maximum(ty1, py1)
        ix2 = jnp.minimum(tx2, px2)
        iy2 = jnp.minimum(ty2, py2)
        iw = jnp.clip(ix2 - ix1, 0.0, None)
        ih = jnp.clip(iy2 - iy1, 0.0, None)
        inter = iw * ih
        ov = inter / (t_area + p_area - inter)
        ov = jnp.where(real, ov, -1.0)
        bigcol = jnp.where(ov == jnp.max(ov, axis=1, keepdims=True),
                           jnp.broadcast_to(col, ov.shape), _PADN)
        best_prior_idx = jnp.min(bigcol, axis=1, keepdims=True)
        trow = lax.broadcasted_iota(jnp.int32, (_NOBJ, 1), 0)
        ovmax0 = jnp.max(ov, axis=0, keepdims=True)
        bigrow = jnp.where(ov == ovmax0,
                           jnp.broadcast_to(trow, ov.shape), _NOBJ)
        best_truth_idx = jnp.min(bigrow, axis=0, keepdims=True)
        best_truth_overlap = ovmax0
        for t in range(_NOBJ):
            hit = col == best_prior_idx[t, 0]
            best_truth_overlap = jnp.where(hit, 2.0, best_truth_overlap)
            best_truth_idx = jnp.where(hit, t, best_truth_idx)
        mx1 = jnp.zeros((1, _PADN), jnp.float32)
        my1 = jnp.zeros((1, _PADN), jnp.float32)
        mx2 = jnp.zeros((1, _PADN), jnp.float32)
        my2 = jnp.zeros((1, _PADN), jnp.float32)
        mlab = jnp.zeros((1, _PADN), jnp.float32)
        for t in range(_NOBJ):
            sel = best_truth_idx == t
            mx1 = jnp.where(sel, tx1[t, 0], mx1)
            my1 = jnp.where(sel, ty1[t, 0], my1)
            mx2 = jnp.where(sel, tx2[t, 0], mx2)
            my2 = jnp.where(sel, ty2[t, 0], my2)
            mlab = jnp.where(sel, tlab[t, 0], mlab)
        conf_t = jnp.where(best_truth_overlap < _OVERLAP_THRESH, 0.0,
                           mlab + 1.0)
        g_cx = ((mx1 + mx2) * 0.5 - cx) / (_VAR0 * pw)
        g_cy = ((my1 + my2) * 0.5 - cy) / (_VAR0 * ph)
        safe_w = jnp.where(real, (mx2 - mx1) / pw, 1.0)
        safe_h = jnp.where(real, (my2 - my1) / ph, 1.0)
        g_w = jnp.log(safe_w) / _VAR1
        g_h = jnp.log(safe_h) / _VAR1

        l0 = locT_ref[i, 0:1, :]
        l1 = locT_ref[i, 1:2, :]
        l2 = locT_ref[i, 2:3, :]
        l3 = locT_ref[i, 3:4, :]
        dcx = cx + l0 * _VAR0 * pw
        dcy = cy + l1 * _VAR0 * ph
        dw = pw * jnp.exp(l2 * _VAR1)
        dh = ph * jnp.exp(l3 * _VAR1)
        dx1 = dcx - dw * 0.5
        dy1 = dcy - dh * 0.5
        dx2 = dx1 + dw
        dy2 = dy1 + dh

        for c in range(1, _C):
            srow = confT_ref[i, c:c + 1, :] * realf
            scores_ref[i * _NCL + (c - 1), :] = srow[0]

        cols = [jnp.full((1, _PADN), float(i), jnp.float32),
                dx1, dy1, dx2, dy2,
                l0, l1, l2, l3]
        for c in range(_C):
            cols.append(confT_ref[i, c:c + 1, :])
        cols += [g_cx, g_cy, g_w, g_h, conf_t]
        while len(cols) < _COLS:
            cols.append(jnp.zeros((1, _PADN), jnp.float32))
        tab = jnp.concatenate([v * realf for v in cols], axis=0)
        payload_ref[i, :, 0:_COLS] = jnp.transpose(tab, (1, 0))
        payload_ref[i, :, _COLS:_PCOLS] = jnp.zeros(
            (_PADN, _PCOLS - _COLS), jnp.float32)


def _tc_prep(loc_data, conf_data, prior_data, targets):
    locT = jnp.pad(jnp.transpose(loc_data, (0, 2, 1)),
                   ((0, 0), (0, 0), (0, _PADN - _NP)))
    conf3 = conf_data.reshape(_B, _NP, _C)
    confT = jnp.pad(jnp.transpose(conf3, (0, 2, 1)),
                    ((0, 0), (0, 0), (0, _PADN - _NP)))
    priorsT = jnp.pad(prior_data.T, ((0, 0), (0, _PADN - _NP)),
                      constant_values=1.0)
    padmask = jnp.arange(_PADN)[None, :] >= _NP
    priorsT = jnp.where(padmask & (jnp.arange(4)[:, None] < 2), 100.0, priorsT)
    scores, payload = pl.pallas_call(
        _tc_prep_body,
        out_shape=[
            jax.ShapeDtypeStruct((_B * _NCL, _PADN), jnp.float32),
            jax.ShapeDtypeStruct((_B, _PADN, _PCOLS), jnp.float32),
        ],
    )(locT, confT, priorsT, targets)
    return scores, payload.reshape(_B * _PADN, _PCOLS)


_CBUF = 512
_CCAP = _CBUF - 16
_MRG = 1024
_NVEC = _PADN // 16


def _sc_select_body(scores_hbm, payload_hbm, out_hbm,
                    sv, cb_key, cb_idx, hist, cand_rows, bx, keep,
                    m_key, m_prior, m_keep, seen, outp, shared, sem):
    c = lax.axis_index("c")
    s = lax.axis_index("s")
    lane = lax.iota(jnp.int32, 16)
    ones16 = jnp.ones((16,), jnp.int32)
    zero16 = jnp.zeros((16,), jnp.int32)

    def unrolled(n, unroll, fn):
        def body(i, _):
            for u in range(unroll):
                fn(i * unroll + u)
            return 0
        lax.fori_loop(0, n // unroll, body, 0)
        for j in range(n - n % unroll, n):
            fn(j)

    def hist_zero():
        unrolled(256, 4,
                 lambda j: plsc.store_scatter(hist, [j * 16 + lane], zero16))

    def hist_pass(bucket_fn):
        def body(j):
            b, m = bucket_fn(j)
            plsc.addupdate_scatter(hist, [lane * 256 + b], ones16, mask=m)
        unrolled(_NVEC, 4, body)

    def hist_select(rank):
        run = jnp.int32(0)
        B = jnp.int32(0)
        ca = jnp.int32(0)
        found = jnp.bool_(False)
        for g in range(15, -1, -1):
            acc = zero16
            for l in range(16):
                acc = acc + hist[pl.ds(l * 256 + g * 16, 16)]
            sfx = lax.rev(plsc.cumsum(lax.rev(acc, (0,))), (0,)) + run
            mask = sfx >= rank
            cnt = plsc.all_reduce_population_count(mask)[0]
            cav = jnp.sum(jnp.where(lane == cnt, sfx, 0))
            ca_g = jnp.where(cnt == 16, run, cav)
            hit = jnp.logical_and(jnp.logical_not(found), cnt > 0)
            B = jnp.where(hit, g * 16 + cnt - 1, B)
            ca = jnp.where(hit, ca_g, ca)
            found = jnp.logical_or(found, cnt > 0)
            run = run + jnp.sum(acc)
        return B, ca, found

    def bitonic(keyref, valref, n):
        nb = n // 16

        def vsort_sweep(kk):
            k16 = kk // 16
            def body(b, _):
                base = b * 16
                key = plsc.load_gather(keyref, [base + lane])
                val = plsc.load_gather(valref, [base + lane])
                desc = (b & k16) == 0
                tkey = jnp.where(desc, key, -1 - key)
                skey, sval = plsc.sort_key_val(tkey, val, descending=True)
                skey = jnp.where(desc, skey, -1 - skey)
                plsc.store_scatter(keyref, [base + lane], skey)
                plsc.store_scatter(valref, [base + lane], sval)
                return 0
            lax.fori_loop(0, nb, body, 0)

        def cross_sweep(kk, j):
            j16 = j // 16
            k16 = kk // 16
            p = j16.bit_length() - 1
            def body(t, _):
                b = lax.shift_left(lax.shift_right_logical(t, p), p + 1) \
                    + jnp.bitwise_and(t, j16 - 1)
                if True:
                    base_a = b * 16
                    base_b = (b + j16) * 16
                    ka = plsc.load_gather(keyref, [base_a + lane])
                    va = plsc.load_gather(valref, [base_a + lane])
                    kb = plsc.load_gather(keyref, [base_b + lane])
                    vb = plsc.load_gather(valref, [base_b + lane])
                    desc = (b & k16) == 0
                    swap = jnp.where(desc, ka < kb, ka > kb)
                    plsc.store_scatter(keyref, [base_a + lane],
                                       jnp.where(swap, kb, ka))
                    plsc.store_scatter(keyref, [base_b + lane],
                                       jnp.where(swap, ka, kb))
                    plsc.store_scatter(valref, [base_a + lane],
                                       jnp.where(swap, vb, va))
                    plsc.store_scatter(valref, [base_b + lane],
                                       jnp.where(swap, va, vb))
                return 0
            lax.fori_loop(0, nb // 2, body, 0)

        vsort_sweep(16)
        kk = 32
        while kk <= n:
            j = kk // 2
            while j >= 16:
                cross_sweep(kk, j)
                j //= 2
            vsort_sweep(kk)
            kk *= 2

    @pl.when(s < _NCL)
    def _phase1():
        row = c * _NCL + s
        pltpu.sync_copy(scores_hbm.at[pl.ds(row * _PADN, _PADN)], sv)

        def load_chunk(j):
            v = plsc.load_gather(sv, [j * 16 + lane])
            m = v > _CONF_THRESH
            k = plsc.bitcast(v, jnp.int32)
            return v, m, k

        hist_zero()
        def bucket_a(j):
            _, m, k = load_chunk(j)
            return lax.shift_right_logical(k, 23), m
        hist_pass(bucket_a)
        b1, ca1, found1 = hist_select(jnp.int32(_TOPK))

        hist_zero()
        def bucket_b(j):
            _, m, k = load_chunk(j)
            m2 = jnp.logical_and(m, lax.shift_right_logical(k, 23) == b1)
            return jnp.bitwise_and(lax.shift_right_logical(k, 15), 255), m2
        hist_pass(bucket_b)
        b2, _, _ = hist_select(_TOPK - ca1)
        lo = jnp.where(found1,
                       jnp.bitwise_or(lax.shift_left(b1, 23),
                                      lax.shift_left(b2, 15)),
                       jnp.int32(0))

        def initcb(j):
            plsc.store_scatter(cb_key, [j * 16 + lane], zero16)
            plsc.store_scatter(cb_idx, [j * 16 + lane],
                               jnp.full((16,), _ZROW, jnp.int32))
        unrolled(_CBUF // 16, 4, initcb)

        def coll(j, off):
            idxv = j * 16 + lane
            v = plsc.load_gather(sv, [idxv])
            m = v > _CONF_THRESH
            k = plsc.bitcast(v, jnp.int32)
            cm = jnp.logical_and(m, k >= lo)
            cmi = cm.astype(jnp.int32)
            pos = off + plsc.cumsum(cmi) - 1
            guard = jnp.logical_and(cm, pos < _CCAP)
            plsc.store_scatter(cb_key, [pos], k, mask=guard)
            plsc.store_scatter(cb_idx, [pos], idxv, mask=guard)
            return jnp.minimum(off + jnp.sum(cmi), _CCAP)
        def coll4(i, off):
            for u in range(4):
                off = coll(i * 4 + u, off)
            return off
        off = lax.fori_loop(0, _NVEC // 4, coll4, jnp.int32(0))

        @pl.when(off <= 128)
        def _small():
            bitonic(cb_key, cb_idx, 128)

        @pl.when(off > 128)
        def _big():
            bitonic(cb_key, cb_idx, _CBUF)

        for j in range(_CAND // 16):
            iv = plsc.load_gather(cb_idx, [j * 16 + lane])
            plsc.store_scatter(outp, [j * 16 + lane], iv + c * _PADN)
        pltpu.async_copy(payload_hbm.at[outp], cand_rows, sem).wait()

        def getcol(j):
            r = j * 16 + lane
            x1 = plsc.load_gather(cand_rows, [r, jnp.full((16,), 1, jnp.int32)])
            y1 = plsc.load_gather(cand_rows, [r, jnp.full((16,), 2, jnp.int32)])
            x2 = plsc.load_gather(cand_rows, [r, jnp.full((16,), 3, jnp.int32)])
            y2 = plsc.load_gather(cand_rows, [r, jnp.full((16,), 4, jnp.int32)])
            plsc.store_scatter(bx, [0 * _CAND + r], x1)
            plsc.store_scatter(bx, [1 * _CAND + r], y1)
            plsc.store_scatter(bx, [2 * _CAND + r], x2)
            plsc.store_scatter(bx, [3 * _CAND + r], y2)
            plsc.store_scatter(bx, [4 * _CAND + r], (x2 - x1) * (y2 - y1))
            plsc.store_scatter(keep, [r], zero16)
        for j in range(_CAND // 16):
            getcol(j)

        def nms_step(i, _):
            i16 = jnp.full((16,), i, jnp.int32)
            xx1 = plsc.load_gather(bx, [0 * _CAND + i16])
            yy1 = plsc.load_gather(bx, [1 * _CAND + i16])
            xx2 = plsc.load_gather(bx, [2 * _CAND + i16])
            yy2 = plsc.load_gather(bx, [3 * _CAND + i16])
            aar = plsc.load_gather(bx, [4 * _CAND + i16])
            key_i = plsc.load_gather(cb_key, [i16])[0]
            def chunk(j, acc):
                r = j * 16 + lane
                x1 = plsc.load_gather(bx, [0 * _CAND + r])
                y1 = plsc.load_gather(bx, [1 * _CAND + r])
                x2 = plsc.load_gather(bx, [2 * _CAND + r])
                y2 = plsc.load_gather(bx, [3 * _CAND + r])
                ar = plsc.load_gather(bx, [4 * _CAND + r])
                kp = plsc.load_gather(keep, [r])
                iw = jnp.maximum(jnp.minimum(x2, xx2) - jnp.maximum(x1, xx1),
                                 0.0)
                ih = jnp.maximum(jnp.minimum(y2, yy2) - jnp.maximum(y1, yy1),
                                 0.0)
                inter = iw * ih
                iou = inter / (ar + aar - inter)
                hit = jnp.logical_and(kp > 0, iou > _NMS_THRESH)
                return jnp.logical_or(acc, hit)
            accv = lax.fori_loop(0, lax.shift_right_logical(i, 4) + 1, chunk,
                                 jnp.zeros((16,), jnp.bool_))
            sup = jnp.any(accv)
            kv = jnp.logical_and(key_i > _BITS001,
                                 jnp.logical_not(sup)).astype(jnp.int32)
            plsc.store_scatter(keep, [i16], jnp.full((16,), kv, jnp.int32),
                               mask=lane == 0)
            return 0
        lax.fori_loop(0, _TOPK, nms_step, 0)

        pltpu.sync_copy(cb_idx.at[pl.ds(0, _CAND)], shared.at[s, 0])
        pltpu.sync_copy(cb_key.at[pl.ds(0, _CAND)], shared.at[s, 1])
        pltpu.sync_copy(keep, shared.at[s, 2])

    plsc.subcore_barrier()

    @pl.when(s == _NCL)
    def _phase2():
        for u in range(_NCL):
            pltpu.sync_copy(shared.at[u, 0], m_prior.at[pl.ds(u * _CAND, _CAND)])
            pltpu.sync_copy(shared.at[u, 1], m_key.at[pl.ds(u * _CAND, _CAND)])
            pltpu.sync_copy(shared.at[u, 2], m_keep.at[pl.ds(u * _CAND, _CAND)])
        unrolled(_NVEC, 4,
                 lambda j: plsc.store_scatter(seen, [j * 16 + lane], zero16))
        for u in range(_NCL):
            for j in range(_CAND // 16):
                sl = u * _CAND + j * 16 + lane
                p = plsc.load_gather(m_prior, [sl])
                kp = plsc.load_gather(m_keep, [sl]) > 0
                dup = plsc.load_gather(seen, [p]) > 0
                k = plsc.load_gather(m_key, [sl])
                newk = jnp.where(
                    jnp.logical_and(kp, jnp.logical_not(dup)), k, 0)
                plsc.store_scatter(m_key, [sl], newk)
                plsc.store_scatter(seen, [p], ones16, mask=kp)
        def initcb2(j):
            plsc.store_scatter(cb_key, [j * 16 + lane], zero16)
            plsc.store_scatter(cb_idx, [j * 16 + lane],
                               jnp.full((16,), _ZROW, jnp.int32))
        unrolled(_CBUF // 16, 4, initcb2)

        def compress(j, off):
            sl = j * 16 + lane
            k = plsc.load_gather(m_key, [sl])
            p = plsc.load_gather(m_prior, [sl])
            m = k > _BITS001
            mi = m.astype(jnp.int32)
            pos = off + plsc.cumsum(mi) - 1
            plsc.store_scatter(cb_key, [pos], k, mask=m)
            plsc.store_scatter(cb_idx, [pos], p, mask=m)
            return off + jnp.sum(mi)
        def compress5(i, off):
            for u in range(5):
                off = compress(i * 5 + u, off)
            return off
        lax.fori_loop(0, (_NCL * _CAND) // 80, compress5, jnp.int32(0))

        bitonic(cb_key, cb_idx, _CBUF)
        for j in range(_CAND // 16):
            r = j * 16 + lane
            k = plsc.load_gather(cb_key, [r])
            p = plsc.load_gather(cb_idx, [r])
            o = jnp.where(k > _BITS001, p, _ZROW) + c * _PADN
            plsc.store_scatter(outp, [r], o)
        pltpu.async_copy(payload_hbm.at[outp], cand_rows, sem).wait()
        pltpu.sync_copy(cand_rows, out_hbm.at[c])


def _sc_select(scores, payload):
    mesh = plsc.VectorSubcoreMesh(core_axis_name="c", subcore_axis_name="s",
                                  num_cores=_B, num_subcores=16)
    f = pl.kernel(
        _sc_select_body,
        out_type=jax.ShapeDtypeStruct((_B, _CAND, _PCOLS), jnp.float32),
        mesh=mesh,
        compiler_params=pltpu.CompilerParams(needs_layout_passes=False),
        scratch_types=[
            pltpu.VMEM((_PADN,), jnp.float32),
            pltpu.VMEM((_CBUF,), jnp.int32),
            pltpu.VMEM((_CBUF,), jnp.int32),
            pltpu.VMEM((4096,), jnp.int32),
            pltpu.VMEM((_CAND, _PCOLS), jnp.float32),
            pltpu.VMEM((5 * _CAND,), jnp.float32),
            pltpu.VMEM((_CAND,), jnp.int32),
            pltpu.VMEM((_MRG,), jnp.int32),
            pltpu.VMEM((_MRG,), jnp.int32),
            pltpu.VMEM((_MRG,), jnp.int32),
            pltpu.VMEM((_PADN,), jnp.int32),
            pltpu.VMEM((_CAND,), jnp.int32),
            pltpu.VMEM_SHARED((16, 3, _CAND), jnp.int32),
            pltpu.SemaphoreType.DMA,
        ],
    )
    return f(scores.reshape(-1), payload)


def _select_jax(scores, payload):
    outs = []
    for i in range(_B):
        keys_all, prior_all, kept_all = [], [], []
        for c in range(_NCL):
            s = scores[i * _NCL + c]
            masked = jnp.where(s > _CONF_THRESH, s, -jnp.inf)
            _, cand = lax.top_k(masked, _TOPK)
            sv = s[cand]
            valid = sv > _CONF_THRESH
            rows = payload[i * _PADN + cand]
            x1, y1, x2, y2 = rows[:, 1], rows[:, 2], rows[:, 3], rows[:, 4]
            area = (x2 - x1) * (y2 - y1)
            ix1 = jnp.maximum(x1[:, None], x1[None, :])
            iy1 = jnp.maximum(y1[:, None], y1[None, :])
            ix2 = jnp.minimum(x2[:, None], x2[None, :])
            iy2 = jnp.minimum(y2[:, None], y2[None, :])
            iw = jnp.clip(ix2 - ix1, 0.0, None)
            ih = jnp.clip(iy2 - iy1, 0.0, None)
            inter = iw * ih
            iou = inter / (area[:, None] + area[None, :] - inter)
            keep = jnp.zeros((_TOPK,), bool)
            def nms_step(k, keep):
                sup = jnp.any(keep & (iou[:, k] > _NMS_THRESH))
                return keep.at[k].set(valid[k] & jnp.logical_not(sup))
            keep = lax.fori_loop(0, _TOPK, nms_step, keep)
            keys_all.append(jnp.where(keep, sv, 0.0))
            prior_all.append(cand)
            kept_all.append(keep)
        keys = jnp.concatenate(keys_all)
        prior = jnp.concatenate(prior_all)
        kept = jnp.concatenate(kept_all)
        M = keys.shape[0]
        same = prior[:, None] == prior[None, :]
        earlier = jnp.arange(M)[None, :] < jnp.arange(M)[:, None]
        dup = jnp.any(same & earlier & kept[None, :], axis=1)
        final = kept & jnp.logical_not(dup)
        key_bits = jnp.where(final, keys, 0.0)
        order = jnp.argsort(-key_bits)[:_TOPK]
        sel_prior = jnp.where(key_bits[order] > _CONF_THRESH,
                              prior[order], _ZROW)
        outs.append(payload[i * _PADN + sel_prior])
    return jnp.stack(outs)


def kernel(loc_data, conf_data, prior_data, targets):
    scores, payload = _tc_prep(loc_data, conf_data, prior_data, targets)
    result = _sc_select(scores, payload)[:, :_TOPK, :]
    rois = result[..., 0:5]
    loc = result[..., 5:9]
    cls = result[..., 9:9 + _C]
    loc_truth = result[..., 9 + _C:13 + _C]
    conf_truth = result[..., 13 + _C:14 + _C]
    return rois, loc, cls, loc_truth, conf_truth


# speedup vs baseline: 40.1514x; 1.0499x over previous
import functools
import struct

import jax
import jax.numpy as jnp
from jax import lax
from jax.experimental import pallas as pl
from jax.experimental.pallas import tpu as pltpu
from jax.experimental.pallas import tpu_sc as plsc

_C = 6
_NCL = _C - 1
_B = 2
_NP = 5000
_PADN = 5120
_TOPK = 100
_CAND = 112
_OVERLAP_THRESH = 0.5
_CONF_THRESH = 0.01
_NMS_THRESH = 0.45
_VAR0 = 0.1
_VAR1 = 0.2
_NOBJ = 8
_COLS = 24
_PCOLS = 128
_ZROW = _NP
_BITS001 = struct.unpack("<i", struct.pack("<f", _CONF_THRESH))[0]


def _tc_prep_body(locT_ref, confT_ref, priorsT_ref, targets_ref,
                  scores_ref, payload_ref):
    cx = priorsT_ref[0:1, :]
    cy = priorsT_ref[1:2, :]
    pw = priorsT_ref[2:3, :]
    ph = priorsT_ref[3:4, :]
    col = lax.broadcasted_iota(jnp.int32, (1, _PADN), 1)
    real = col < _NP
    realf = real.astype(jnp.float32)
    px1 = cx - pw * 0.5
    py1 = cy - ph * 0.5
    px2 = cx + pw * 0.5
    py2 = cy + ph * 0.5
    p_area = (px2 - px1) * (py2 - py1)

    for i in range(_B):
        tx1 = targets_ref[i, :, 0:1]
        ty1 = targets_ref[i, :, 1:2]
        tx2 = targets_ref[i, :, 2:3]
        ty2 = targets_ref[i, :, 3:4]
        tlab = targets_ref[i, :, 4:5]
        t_area = (tx2 - tx1) * (ty2 - ty1)
        ix1 = jnp.maximum(tx1, px1)
        iy1 = jnp.maximum(ty1, py1)
        ix2 = jnp.minimum(tx2, px2)
        iy2 = jnp.minimum(ty2, py2)
        iw = jnp.clip(ix2 - ix1, 0.0, None)
        ih = jnp.clip(iy2 - iy1, 0.0, None)
        inter = iw * ih
        ov = inter / (t_area + p_area - inter)
        ov = jnp.where(real, ov, -1.0)
        bigcol = jnp.where(ov == jnp.max(ov, axis=1, keepdims=True),
                           jnp.broadcast_to(col, ov.shape), _PADN)
        best_prior_idx = jnp.min(bigcol, axis=1, keepdims=True)
        trow = lax.broadcasted_iota(jnp.int32, (_NOBJ, 1), 0)
        ovmax0 = jnp.max(ov, axis=0, keepdims=True)
        bigrow = jnp.where(ov == ovmax0,
                           jnp.broadcast_to(trow, ov.shape), _NOBJ)
        best_truth_idx = jnp.min(bigrow, axis=0, keepdims=True)
        best_truth_overlap = ovmax0
        for t in range(_NOBJ):
            hit = col == best_prior_idx[t, 0]
            best_truth_overlap = jnp.where(hit, 2.0, best_truth_overlap)
            best_truth_idx = jnp.where(hit, t, best_truth_idx)
        mx1 = jnp.zeros((1, _PADN), jnp.float32)
        my1 = jnp.zeros((1, _PADN), jnp.float32)
        mx2 = jnp.zeros((1, _PADN), jnp.float32)
        my2 = jnp.zeros((1, _PADN), jnp.float32)
        mlab = jnp.zeros((1, _PADN), jnp.float32)
        for t in range(_NOBJ):
            sel = best_truth_idx == t
            mx1 = jnp.where(sel, tx1[t, 0], mx1)
            my1 = jnp.where(sel, ty1[t, 0], my1)
            mx2 = jnp.where(sel, tx2[t, 0], mx2)
            my2 = jnp.where(sel, ty2[t, 0], my2)
            mlab = jnp.where(sel, tlab[t, 0], mlab)
        conf_t = jnp.where(best_truth_overlap < _OVERLAP_THRESH, 0.0,
                           mlab + 1.0)
        g_cx = ((mx1 + mx2) * 0.5 - cx) / (_VAR0 * pw)
        g_cy = ((my1 + my2) * 0.5 - cy) / (_VAR0 * ph)
        safe_w = jnp.where(real, (mx2 - mx1) / pw, 1.0)
        safe_h = jnp.where(real, (my2 - my1) / ph, 1.0)
        g_w = jnp.log(safe_w) / _VAR1
        g_h = jnp.log(safe_h) / _VAR1

        l0 = locT_ref[i, 0:1, :]
        l1 = locT_ref[i, 1:2, :]
        l2 = locT_ref[i, 2:3, :]
        l3 = locT_ref[i, 3:4, :]
        dcx = cx + l0 * _VAR0 * pw
        dcy = cy + l1 * _VAR0 * ph
        dw = pw * jnp.exp(l2 * _VAR1)
        dh = ph * jnp.exp(l3 * _VAR1)
        dx1 = dcx - dw * 0.5
        dy1 = dcy - dh * 0.5
        dx2 = dx1 + dw
        dy2 = dy1 + dh

        for c in range(1, _C):
            srow = confT_ref[i, c:c + 1, :] * realf
            scores_ref[i * _NCL + (c - 1), :] = srow[0]

        cols = [jnp.full((1, _PADN), float(i), jnp.float32),
                dx1, dy1, dx2, dy2,
                l0, l1, l2, l3]
        for c in range(_C):
            cols.append(confT_ref[i, c:c + 1, :])
        cols += [g_cx, g_cy, g_w, g_h, conf_t]
        while len(cols) < _COLS:
            cols.append(jnp.zeros((1, _PADN), jnp.float32))
        tab = jnp.concatenate([v * realf for v in cols], axis=0)
        payload_ref[i, :, 0:_COLS] = jnp.transpose(tab, (1, 0))
        payload_ref[i, :, _COLS:_PCOLS] = jnp.zeros(
            (_PADN, _PCOLS - _COLS), jnp.float32)


def _tc_prep(loc_data, conf_data, prior_data, targets):
    locT = jnp.pad(jnp.transpose(loc_data, (0, 2, 1)),
                   ((0, 0), (0, 0), (0, _PADN - _NP)))
    conf3 = conf_data.reshape(_B, _NP, _C)
    confT = jnp.pad(jnp.transpose(conf3, (0, 2, 1)),
                    ((0, 0), (0, 0), (0, _PADN - _NP)))
    priorsT = jnp.pad(prior_data.T, ((0, 0), (0, _PADN - _NP)),
                      constant_values=1.0)
    padmask = jnp.arange(_PADN)[None, :] >= _NP
    priorsT = jnp.where(padmask & (jnp.arange(4)[:, None] < 2), 100.0, priorsT)
    scores, payload = pl.pallas_call(
        _tc_prep_body,
        out_shape=[
            jax.ShapeDtypeStruct((_B * _NCL, _PADN), jnp.float32),
            jax.ShapeDtypeStruct((_B, _PADN, _PCOLS), jnp.float32),
        ],
    )(locT, confT, priorsT, targets)
    return scores, payload.reshape(_B * _PADN, _PCOLS)


_CBUF = 512
_CCAP = _CBUF - 16
_MRG = 1024
_NVEC = _PADN // 16


def _sc_select_body(scores_hbm, payload_hbm, out_hbm,
                    sv, cb_key, cb_idx, hist, cand_rows, bx, keep,
                    m_key, m_prior, m_keep, seen, outp, shared, sem):
    c = lax.axis_index("c")
    s = lax.axis_index("s")
    lane = lax.iota(jnp.int32, 16)
    ones16 = jnp.ones((16,), jnp.int32)
    zero16 = jnp.zeros((16,), jnp.int32)

    def unrolled(n, unroll, fn):
        def body(i, _):
            for u in range(unroll):
                fn(i * unroll + u)
            return 0
        lax.fori_loop(0, n // unroll, body, 0)
        for j in range(n - n % unroll, n):
            fn(j)

    def hist_zero():
        unrolled(256, 4,
                 lambda j: plsc.store_scatter(hist, [j * 16 + lane], zero16))

    def hist_pass(bucket_fn):
        def body(j):
            b, m = bucket_fn(j)
            plsc.addupdate_scatter(hist, [lane * 256 + b], ones16, mask=m)
        unrolled(_NVEC, 4, body)

    def hist_select(rank):
        run = jnp.int32(0)
        B = jnp.int32(0)
        ca = jnp.int32(0)
        found = jnp.bool_(False)
        for g in range(15, -1, -1):
            acc = zero16
            for l in range(16):
                acc = acc + hist[pl.ds(l * 256 + g * 16, 16)]
            sfx = lax.rev(plsc.cumsum(lax.rev(acc, (0,))), (0,)) + run
            mask = sfx >= rank
            cnt = plsc.all_reduce_population_count(mask)[0]
            cav = jnp.sum(jnp.where(lane == cnt, sfx, 0))
            ca_g = jnp.where(cnt == 16, run, cav)
            hit = jnp.logical_and(jnp.logical_not(found), cnt > 0)
            B = jnp.where(hit, g * 16 + cnt - 1, B)
            ca = jnp.where(hit, ca_g, ca)
            found = jnp.logical_or(found, cnt > 0)
            run = run + jnp.sum(acc)
        return B, ca, found

    def bitonic(keyref, valref, n):
        nb = n // 16

        def vsort_sweep(kk):
            k16 = kk // 16
            def body(b, _):
                base = b * 16
                key = plsc.load_gather(keyref, [base + lane])
                val = plsc.load_gather(valref, [base + lane])
                desc = (b & k16) == 0
                tkey = jnp.where(desc, key, -1 - key)
                skey, sval = plsc.sort_key_val(tkey, val, descending=True)
                skey = jnp.where(desc, skey, -1 - skey)
                plsc.store_scatter(keyref, [base + lane], skey)
                plsc.store_scatter(valref, [base + lane], sval)
                return 0
            lax.fori_loop(0, nb, body, 0)

        def cross_sweep(kk, j):
            j16 = j // 16
            k16 = kk // 16
            p = j16.bit_length() - 1
            def body(t, _):
                b = lax.shift_left(lax.shift_right_logical(t, p), p + 1) \
                    + jnp.bitwise_and(t, j16 - 1)
                if True:
                    base_a = b * 16
                    base_b = (b + j16) * 16
                    ka = plsc.load_gather(keyref, [base_a + lane])
                    va = plsc.load_gather(valref, [base_a + lane])
                    kb = plsc.load_gather(keyref, [base_b + lane])
                    vb = plsc.load_gather(valref, [base_b + lane])
                    desc = (b & k16) == 0
                    swap = jnp.where(desc, ka < kb, ka > kb)
                    plsc.store_scatter(keyref, [base_a + lane],
                                       jnp.where(swap, kb, ka))
                    plsc.store_scatter(keyref, [base_b + lane],
                                       jnp.where(swap, ka, kb))
                    plsc.store_scatter(valref, [base_a + lane],
                                       jnp.where(swap, vb, va))
                    plsc.store_scatter(valref, [base_b + lane],
                                       jnp.where(swap, va, vb))
                return 0
            lax.fori_loop(0, nb // 2, body, 0)

        vsort_sweep(16)
        kk = 32
        while kk <= n:
            j = kk // 2
            while j >= 16:
                cross_sweep(kk, j)
                j //= 2
            vsort_sweep(kk)
            kk *= 2

    @pl.when(s < _NCL)
    def _phase1():
        row = c * _NCL + s
        pltpu.sync_copy(scores_hbm.at[pl.ds(row * _PADN, _PADN)], sv)

        def load_chunk(j):
            v = plsc.load_gather(sv, [j * 16 + lane])
            m = v > _CONF_THRESH
            k = plsc.bitcast(v, jnp.int32)
            return v, m, k

        hist_zero()
        def bucket_a(j):
            _, m, k = load_chunk(j)
            return lax.shift_right_logical(k, 23), m
        hist_pass(bucket_a)
        b1, ca1, found1 = hist_select(jnp.int32(_TOPK))

        hist_zero()
        def bucket_b(j):
            _, m, k = load_chunk(j)
            m2 = jnp.logical_and(m, lax.shift_right_logical(k, 23) == b1)
            return jnp.bitwise_and(lax.shift_right_logical(k, 15), 255), m2
        hist_pass(bucket_b)
        b2, _, _ = hist_select(_TOPK - ca1)
        lo = jnp.where(found1,
                       jnp.bitwise_or(lax.shift_left(b1, 23),
                                      lax.shift_left(b2, 15)),
                       jnp.int32(0))

        def initcb(j):
            plsc.store_scatter(cb_key, [j * 16 + lane], zero16)
            plsc.store_scatter(cb_idx, [j * 16 + lane],
                               jnp.full((16,), _ZROW, jnp.int32))
        unrolled(_CBUF // 16, 4, initcb)

        def coll(j, off):
            idxv = j * 16 + lane
            v = plsc.load_gather(sv, [idxv])
            m = v > _CONF_THRESH
            k = plsc.bitcast(v, jnp.int32)
            cm = jnp.logical_and(m, k >= lo)
            cmi = cm.astype(jnp.int32)
            pos = off + plsc.cumsum(cmi) - 1
            guard = jnp.logical_and(cm, pos < _CCAP)
            plsc.store_scatter(cb_key, [pos], k, mask=guard)
            plsc.store_scatter(cb_idx, [pos], idxv, mask=guard)
            return jnp.minimum(off + jnp.sum(cmi), _CCAP)
        def coll4(i, off):
            for u in range(4):
                off = coll(i * 4 + u, off)
            return off
        off = lax.fori_loop(0, _NVEC // 4, coll4, jnp.int32(0))

        @pl.when(off <= 128)
        def _small():
            bitonic(cb_key, cb_idx, 128)

        @pl.when(off > 128)
        def _big():
            bitonic(cb_key, cb_idx, _CBUF)

        for j in range(_CAND // 16):
            iv = plsc.load_gather(cb_idx, [j * 16 + lane])
            plsc.store_scatter(outp, [j * 16 + lane], iv + c * _PADN)
        pltpu.async_copy(payload_hbm.at[outp], cand_rows, sem).wait()

        def getcol(j):
            r = j * 16 + lane
            x1 = plsc.load_gather(cand_rows, [r, jnp.full((16,), 1, jnp.int32)])
            y1 = plsc.load_gather(cand_rows, [r, jnp.full((16,), 2, jnp.int32)])
            x2 = plsc.load_gather(cand_rows, [r, jnp.full((16,), 3, jnp.int32)])
            y2 = plsc.load_gather(cand_rows, [r, jnp.full((16,), 4, jnp.int32)])
            plsc.store_scatter(bx, [0 * _CAND + r], x1)
            plsc.store_scatter(bx, [1 * _CAND + r], y1)
            plsc.store_scatter(bx, [2 * _CAND + r], x2)
            plsc.store_scatter(bx, [3 * _CAND + r], y2)
            plsc.store_scatter(bx, [4 * _CAND + r], (x2 - x1) * (y2 - y1))
            plsc.store_scatter(keep, [r], zero16)
        for j in range(_CAND // 16):
            getcol(j)

        nch = _CAND // 16
        bxv = []
        for j in range(nch):
            r = j * 16 + lane
            bxv.append(tuple(plsc.load_gather(bx, [k * _CAND + r])
                             for k in range(5)))

        def nms_step(i, kcarry):
            i16 = jnp.full((16,), i, jnp.int32)
            xx1 = plsc.load_gather(bx, [0 * _CAND + i16])
            yy1 = plsc.load_gather(bx, [1 * _CAND + i16])
            xx2 = plsc.load_gather(bx, [2 * _CAND + i16])
            yy2 = plsc.load_gather(bx, [3 * _CAND + i16])
            aar = plsc.load_gather(bx, [4 * _CAND + i16])
            key_i = plsc.load_gather(cb_key, [i16])[0]
            ichunk = lax.shift_right_logical(i, 4)
            iline = jnp.bitwise_and(i, 15)
            sup = jnp.zeros((16,), jnp.bool_)
            for j in range(nch):
                x1, y1, x2, y2, ar = bxv[j]
                iw = jnp.maximum(jnp.minimum(x2, xx2) - jnp.maximum(x1, xx1),
                                 0.0)
                ih = jnp.maximum(jnp.minimum(y2, yy2) - jnp.maximum(y1, yy1),
                                 0.0)
                inter = iw * ih
                iou = inter / (ar + aar - inter)
                hit = jnp.logical_and(kcarry[j], iou > _NMS_THRESH)
                sup = jnp.logical_or(sup, hit)
            kv = jnp.logical_and(key_i > _BITS001,
                                 jnp.logical_not(jnp.any(sup)))
            hitlane = lane == iline
            return tuple(
                jnp.logical_or(kcarry[j],
                               jnp.logical_and(jnp.logical_and(hitlane, kv),
                                               ichunk == j))
                for j in range(nch))

        kfin = lax.fori_loop(0, _TOPK, nms_step,
                             tuple(jnp.zeros((16,), jnp.bool_)
                                   for _ in range(nch)))
        for j in range(nch):
            plsc.store_scatter(keep, [j * 16 + lane],
                               kfin[j].astype(jnp.int32))

        pltpu.sync_copy(cb_idx.at[pl.ds(0, _CAND)], shared.at[s, 0])
        pltpu.sync_copy(cb_key.at[pl.ds(0, _CAND)], shared.at[s, 1])
        pltpu.sync_copy(keep, shared.at[s, 2])

    plsc.subcore_barrier()

    @pl.when(s == _NCL)
    def _phase2():
        for u in range(_NCL):
            pltpu.sync_copy(shared.at[u, 0], m_prior.at[pl.ds(u * _CAND, _CAND)])
            pltpu.sync_copy(shared.at[u, 1], m_key.at[pl.ds(u * _CAND, _CAND)])
            pltpu.sync_copy(shared.at[u, 2], m_keep.at[pl.ds(u * _CAND, _CAND)])
        unrolled(_NVEC, 4,
                 lambda j: plsc.store_scatter(seen, [j * 16 + lane], zero16))
        for u in range(_NCL):
            for j in range(_CAND // 16):
                sl = u * _CAND + j * 16 + lane
                p = plsc.load_gather(m_prior, [sl])
                kp = plsc.load_gather(m_keep, [sl]) > 0
                dup = plsc.load_gather(seen, [p]) > 0
                k = plsc.load_gather(m_key, [sl])
                newk = jnp.where(
                    jnp.logical_and(kp, jnp.logical_not(dup)), k, 0)
                plsc.store_scatter(m_key, [sl], newk)
                plsc.store_scatter(seen, [p], ones16, mask=kp)
        def initcb2(j):
            plsc.store_scatter(cb_key, [j * 16 + lane], zero16)
            plsc.store_scatter(cb_idx, [j * 16 + lane],
                               jnp.full((16,), _ZROW, jnp.int32))
        unrolled(_CBUF // 16, 4, initcb2)

        def compress(j, off):
            sl = j * 16 + lane
            k = plsc.load_gather(m_key, [sl])
            p = plsc.load_gather(m_prior, [sl])
            m = k > _BITS001
            mi = m.astype(jnp.int32)
            pos = off + plsc.cumsum(mi) - 1
            plsc.store_scatter(cb_key, [pos], k, mask=m)
            plsc.store_scatter(cb_idx, [pos], p, mask=m)
            return off + jnp.sum(mi)
        def compress5(i, off):
            for u in range(5):
                off = compress(i * 5 + u, off)
            return off
        lax.fori_loop(0, (_NCL * _CAND) // 80, compress5, jnp.int32(0))

        bitonic(cb_key, cb_idx, _CBUF)
        for j in range(_CAND // 16):
            r = j * 16 + lane
            k = plsc.load_gather(cb_key, [r])
            p = plsc.load_gather(cb_idx, [r])
            o = jnp.where(k > _BITS001, p, _ZROW) + c * _PADN
            plsc.store_scatter(outp, [r], o)
        pltpu.async_copy(payload_hbm.at[outp], cand_rows, sem).wait()
        pltpu.sync_copy(cand_rows, out_hbm.at[c])


def _sc_select(scores, payload):
    mesh = plsc.VectorSubcoreMesh(core_axis_name="c", subcore_axis_name="s",
                                  num_cores=_B, num_subcores=16)
    f = pl.kernel(
        _sc_select_body,
        out_type=jax.ShapeDtypeStruct((_B, _CAND, _PCOLS), jnp.float32),
        mesh=mesh,
        compiler_params=pltpu.CompilerParams(needs_layout_passes=False),
        scratch_types=[
            pltpu.VMEM((_PADN,), jnp.float32),
            pltpu.VMEM((_CBUF,), jnp.int32),
            pltpu.VMEM((_CBUF,), jnp.int32),
            pltpu.VMEM((4096,), jnp.int32),
            pltpu.VMEM((_CAND, _PCOLS), jnp.float32),
            pltpu.VMEM((5 * _CAND,), jnp.float32),
            pltpu.VMEM((_CAND,), jnp.int32),
            pltpu.VMEM((_MRG,), jnp.int32),
            pltpu.VMEM((_MRG,), jnp.int32),
            pltpu.VMEM((_MRG,), jnp.int32),
            pltpu.VMEM((_PADN,), jnp.int32),
            pltpu.VMEM((_CAND,), jnp.int32),
            pltpu.VMEM_SHARED((16, 3, _CAND), jnp.int32),
            pltpu.SemaphoreType.DMA,
        ],
    )
    return f(scores.reshape(-1), payload)


def _select_jax(scores, payload):
    outs = []
    for i in range(_B):
        keys_all, prior_all, kept_all = [], [], []
        for c in range(_NCL):
            s = scores[i * _NCL + c]
            masked = jnp.where(s > _CONF_THRESH, s, -jnp.inf)
            _, cand = lax.top_k(masked, _TOPK)
            sv = s[cand]
            valid = sv > _CONF_THRESH
            rows = payload[i * _PADN + cand]
            x1, y1, x2, y2 = rows[:, 1], rows[:, 2], rows[:, 3], rows[:, 4]
            area = (x2 - x1) * (y2 - y1)
            ix1 = jnp.maximum(x1[:, None], x1[None, :])
            iy1 = jnp.maximum(y1[:, None], y1[None, :])
            ix2 = jnp.minimum(x2[:, None], x2[None, :])
            iy2 = jnp.minimum(y2[:, None], y2[None, :])
            iw = jnp.clip(ix2 - ix1, 0.0, None)
            ih = jnp.clip(iy2 - iy1, 0.0, None)
            inter = iw * ih
            iou = inter / (area[:, None] + area[None, :] - inter)
            keep = jnp.zeros((_TOPK,), bool)
            def nms_step(k, keep):
                sup = jnp.any(keep & (iou[:, k] > _NMS_THRESH))
                return keep.at[k].set(valid[k] & jnp.logical_not(sup))
            keep = lax.fori_loop(0, _TOPK, nms_step, keep)
            keys_all.append(jnp.where(keep, sv, 0.0))
            prior_all.append(cand)
            kept_all.append(keep)
        keys = jnp.concatenate(keys_all)
        prior = jnp.concatenate(prior_all)
        kept = jnp.concatenate(kept_all)
        M = keys.shape[0]
        same = prior[:, None] == prior[None, :]
        earlier = jnp.arange(M)[None, :] < jnp.arange(M)[:, None]
        dup = jnp.any(same & earlier & kept[None, :], axis=1)
        final = kept & jnp.logical_not(dup)
        key_bits = jnp.where(final, keys, 0.0)
        order = jnp.argsort(-key_bits)[:_TOPK]
        sel_prior = jnp.where(key_bits[order] > _CONF_THRESH,
                              prior[order], _ZROW)
        outs.append(payload[i * _PADN + sel_prior])
    return jnp.stack(outs)


def kernel(loc_data, conf_data, prior_data, targets):
    scores, payload = _tc_prep(loc_data, conf_data, prior_data, targets)
    result = _sc_select(scores, payload)[:, :_TOPK, :]
    rois = result[..., 0:5]
    loc = result[..., 5:9]
    cls = result[..., 9:9 + _C]
    loc_truth = result[..., 9 + _C:13 + _C]
    conf_truth = result[..., 13 + _C:14 + _C]
    return rois, loc, cls, loc_truth, conf_truth


# speedup vs baseline: 41.5948x; 1.0359x over previous
import functools
import struct

import jax
import jax.numpy as jnp
from jax import lax
from jax.experimental import pallas as pl
from jax.experimental.pallas import tpu as pltpu
from jax.experimental.pallas import tpu_sc as plsc

_C = 6
_NCL = _C - 1
_B = 2
_NP = 5000
_PADN = 5120
_TOPK = 100
_CAND = 112
_OVERLAP_THRESH = 0.5
_CONF_THRESH = 0.01
_NMS_THRESH = 0.45
_VAR0 = 0.1
_VAR1 = 0.2
_NOBJ = 8
_COLS = 24
_PCOLS = 128
_ZROW = _NP
_BITS001 = struct.unpack("<i", struct.pack("<f", _CONF_THRESH))[0]


def _tc_prep_body(locT_ref, confT_ref, priorsT_ref, targets_ref,
                  scores_ref, payload_ref):
    cx = priorsT_ref[0:1, :]
    cy = priorsT_ref[1:2, :]
    pw = priorsT_ref[2:3, :]
    ph = priorsT_ref[3:4, :]
    col = lax.broadcasted_iota(jnp.int32, (1, _PADN), 1)
    real = col < _NP
    realf = real.astype(jnp.float32)
    px1 = cx - pw * 0.5
    py1 = cy - ph * 0.5
    px2 = cx + pw * 0.5
    py2 = cy + ph * 0.5
    p_area = (px2 - px1) * (py2 - py1)

    for i in range(_B):
        tx1 = targets_ref[i, :, 0:1]
        ty1 = targets_ref[i, :, 1:2]
        tx2 = targets_ref[i, :, 2:3]
        ty2 = targets_ref[i, :, 3:4]
        tlab = targets_ref[i, :, 4:5]
        t_area = (tx2 - tx1) * (ty2 - ty1)
        ix1 = jnp.maximum(tx1, px1)
        iy1 = jnp.maximum(ty1, py1)
        ix2 = jnp.minimum(tx2, px2)
        iy2 = jnp.minimum(ty2, py2)
        iw = jnp.clip(ix2 - ix1, 0.0, None)
        ih = jnp.clip(iy2 - iy1, 0.0, None)
        inter = iw * ih
        ov = inter / (t_area + p_area - inter)
        ov = jnp.where(real, ov, -1.0)
        bigcol = jnp.where(ov == jnp.max(ov, axis=1, keepdims=True),
                           jnp.broadcast_to(col, ov.shape), _PADN)
        best_prior_idx = jnp.min(bigcol, axis=1, keepdims=True)
        trow = lax.broadcasted_iota(jnp.int32, (_NOBJ, 1), 0)
        ovmax0 = jnp.max(ov, axis=0, keepdims=True)
        bigrow = jnp.where(ov == ovmax0,
                           jnp.broadcast_to(trow, ov.shape), _NOBJ)
        best_truth_idx = jnp.min(bigrow, axis=0, keepdims=True)
        best_truth_overlap = ovmax0
        for t in range(_NOBJ):
            hit = col == best_prior_idx[t, 0]
            best_truth_overlap = jnp.where(hit, 2.0, best_truth_overlap)
            best_truth_idx = jnp.where(hit, t, best_truth_idx)
        mx1 = jnp.zeros((1, _PADN), jnp.float32)
        my1 = jnp.zeros((1, _PADN), jnp.float32)
        mx2 = jnp.zeros((1, _PADN), jnp.float32)
        my2 = jnp.zeros((1, _PADN), jnp.float32)
        mlab = jnp.zeros((1, _PADN), jnp.float32)
        for t in range(_NOBJ):
            sel = best_truth_idx == t
            mx1 = jnp.where(sel, tx1[t, 0], mx1)
            my1 = jnp.where(sel, ty1[t, 0], my1)
            mx2 = jnp.where(sel, tx2[t, 0], mx2)
            my2 = jnp.where(sel, ty2[t, 0], my2)
            mlab = jnp.where(sel, tlab[t, 0], mlab)
        conf_t = jnp.where(best_truth_overlap < _OVERLAP_THRESH, 0.0,
                           mlab + 1.0)
        g_cx = ((mx1 + mx2) * 0.5 - cx) / (_VAR0 * pw)
        g_cy = ((my1 + my2) * 0.5 - cy) / (_VAR0 * ph)
        safe_w = jnp.where(real, (mx2 - mx1) / pw, 1.0)
        safe_h = jnp.where(real, (my2 - my1) / ph, 1.0)
        g_w = jnp.log(safe_w) / _VAR1
        g_h = jnp.log(safe_h) / _VAR1

        l0 = locT_ref[i, 0:1, :]
        l1 = locT_ref[i, 1:2, :]
        l2 = locT_ref[i, 2:3, :]
        l3 = locT_ref[i, 3:4, :]
        dcx = cx + l0 * _VAR0 * pw
        dcy = cy + l1 * _VAR0 * ph
        dw = pw * jnp.exp(l2 * _VAR1)
        dh = ph * jnp.exp(l3 * _VAR1)
        dx1 = dcx - dw * 0.5
        dy1 = dcy - dh * 0.5
        dx2 = dx1 + dw
        dy2 = dy1 + dh

        for c in range(1, _C):
            srow = confT_ref[i, c:c + 1, :] * realf
            scores_ref[i * _NCL + (c - 1), :] = srow[0]

        cols = [jnp.full((1, _PADN), float(i), jnp.float32),
                dx1, dy1, dx2, dy2,
                l0, l1, l2, l3]
        for c in range(_C):
            cols.append(confT_ref[i, c:c + 1, :])
        cols += [g_cx, g_cy, g_w, g_h, conf_t]
        while len(cols) < _COLS:
            cols.append(jnp.zeros((1, _PADN), jnp.float32))
        tab = jnp.concatenate([v * realf for v in cols], axis=0)
        payload_ref[i, :, 0:_COLS] = jnp.transpose(tab, (1, 0))
        payload_ref[i, :, _COLS:_PCOLS] = jnp.zeros(
            (_PADN, _PCOLS - _COLS), jnp.float32)


def _tc_prep(loc_data, conf_data, prior_data, targets):
    locT = jnp.pad(jnp.transpose(loc_data, (0, 2, 1)),
                   ((0, 0), (0, 0), (0, _PADN - _NP)))
    conf3 = conf_data.reshape(_B, _NP, _C)
    confT = jnp.pad(jnp.transpose(conf3, (0, 2, 1)),
                    ((0, 0), (0, 0), (0, _PADN - _NP)))
    priorsT = jnp.pad(prior_data.T, ((0, 0), (0, _PADN - _NP)),
                      constant_values=1.0)
    padmask = jnp.arange(_PADN)[None, :] >= _NP
    priorsT = jnp.where(padmask & (jnp.arange(4)[:, None] < 2), 100.0, priorsT)
    scores, payload = pl.pallas_call(
        _tc_prep_body,
        out_shape=[
            jax.ShapeDtypeStruct((_B * _NCL, _PADN), jnp.float32),
            jax.ShapeDtypeStruct((_B, _PADN, _PCOLS), jnp.float32),
        ],
    )(locT, confT, priorsT, targets)
    return scores, payload.reshape(_B * _PADN, _PCOLS)


_CBUF = 512
_CCAP = _CBUF - 16
_MRG = 1024
_NVEC = _PADN // 16


def _sc_select_body(scores_hbm, payload_hbm, out_hbm,
                    sv, cb_key, cb_idx, hist, cand_rows, bx, keep,
                    m_key, m_prior, m_keep, seen, outp, shared, sem):
    c = lax.axis_index("c")
    s = lax.axis_index("s")
    lane = lax.iota(jnp.int32, 16)
    ones16 = jnp.ones((16,), jnp.int32)
    zero16 = jnp.zeros((16,), jnp.int32)

    def unrolled(n, unroll, fn):
        def body(i, _):
            for u in range(unroll):
                fn(i * unroll + u)
            return 0
        lax.fori_loop(0, n // unroll, body, 0)
        for j in range(n - n % unroll, n):
            fn(j)

    def hist_zero():
        unrolled(256, 4,
                 lambda j: plsc.store_scatter(hist, [j * 16 + lane], zero16))

    def hist_pass(bucket_fn):
        def body(j):
            b, m = bucket_fn(j)
            plsc.addupdate_scatter(hist, [lane * 256 + b], ones16, mask=m)
        unrolled(_NVEC, 4, body)

    def hist_select(rank):
        run = jnp.int32(0)
        B = jnp.int32(0)
        ca = jnp.int32(0)
        found = jnp.bool_(False)
        for g in range(15, -1, -1):
            acc = zero16
            for l in range(16):
                acc = acc + hist[pl.ds(l * 256 + g * 16, 16)]
            sfx = lax.rev(plsc.cumsum(lax.rev(acc, (0,))), (0,)) + run
            mask = sfx >= rank
            cnt = plsc.all_reduce_population_count(mask)[0]
            cav = jnp.sum(jnp.where(lane == cnt, sfx, 0))
            ca_g = jnp.where(cnt == 16, run, cav)
            hit = jnp.logical_and(jnp.logical_not(found), cnt > 0)
            B = jnp.where(hit, g * 16 + cnt - 1, B)
            ca = jnp.where(hit, ca_g, ca)
            found = jnp.logical_or(found, cnt > 0)
            run = run + jnp.sum(acc)
        return B, ca, found

    def vsort_sweep(keyref, valref, nb, k16):
        def body(b, _):
            base = b * 16
            key = plsc.load_gather(keyref, [base + lane])
            val = plsc.load_gather(valref, [base + lane])
            desc = (b & k16) == 0
            tkey = jnp.where(desc, key, -1 - key)
            skey, sval = plsc.sort_key_val(tkey, val, descending=True)
            skey = jnp.where(desc, skey, -1 - skey)
            plsc.store_scatter(keyref, [base + lane], skey)
            plsc.store_scatter(valref, [base + lane], sval)
            return 0
        lax.fori_loop(0, nb, body, 0)

    def cross_sweep(keyref, valref, nb, j16, k16):
        p = j16.bit_length() - 1
        def body(t, _):
            b = lax.shift_left(lax.shift_right_logical(t, p), p + 1) \
                + jnp.bitwise_and(t, j16 - 1)
            base_a = b * 16
            base_b = (b + j16) * 16
            ka = plsc.load_gather(keyref, [base_a + lane])
            va = plsc.load_gather(valref, [base_a + lane])
            kb = plsc.load_gather(keyref, [base_b + lane])
            vb = plsc.load_gather(valref, [base_b + lane])
            desc = (b & k16) == 0
            swap = jnp.where(desc, ka < kb, ka > kb)
            plsc.store_scatter(keyref, [base_a + lane],
                               jnp.where(swap, kb, ka))
            plsc.store_scatter(keyref, [base_b + lane],
                               jnp.where(swap, ka, kb))
            plsc.store_scatter(valref, [base_a + lane],
                               jnp.where(swap, vb, va))
            plsc.store_scatter(valref, [base_b + lane],
                               jnp.where(swap, va, vb))
            return 0
        lax.fori_loop(0, nb // 2, body, 0)

    def bitonic(keyref, valref, n):
        nb = n // 16
        vsort_sweep(keyref, valref, nb, 1)
        kk = 32
        while kk <= n:
            j = kk // 2
            while j >= 16:
                cross_sweep(keyref, valref, nb, j // 16, kk // 16)
                j //= 2
            vsort_sweep(keyref, valref, nb, kk // 16)
            kk *= 2

    @pl.when(s < _NCL)
    def _phase1():
        row = c * _NCL + s
        pltpu.sync_copy(scores_hbm.at[pl.ds(row * _PADN, _PADN)], sv)

        def load_chunk(j):
            v = plsc.load_gather(sv, [j * 16 + lane])
            m = v > _CONF_THRESH
            k = plsc.bitcast(v, jnp.int32)
            return v, m, k

        hist_zero()
        def bucket_a(j):
            _, m, k = load_chunk(j)
            return lax.shift_right_logical(k, 23), m
        hist_pass(bucket_a)
        b1, ca1, found1 = hist_select(jnp.int32(_TOPK))

        hist_zero()
        def bucket_b(j):
            _, m, k = load_chunk(j)
            m2 = jnp.logical_and(m, lax.shift_right_logical(k, 23) == b1)
            return jnp.bitwise_and(lax.shift_right_logical(k, 15), 255), m2
        hist_pass(bucket_b)
        b2, _, _ = hist_select(_TOPK - ca1)
        lo = jnp.where(found1,
                       jnp.bitwise_or(lax.shift_left(b1, 23),
                                      lax.shift_left(b2, 15)),
                       jnp.int32(0))

        def initcb(j):
            plsc.store_scatter(cb_key, [j * 16 + lane], zero16)
            plsc.store_scatter(cb_idx, [j * 16 + lane],
                               jnp.full((16,), _ZROW, jnp.int32))
        unrolled(_CBUF // 16, 4, initcb)

        def coll(j, off):
            idxv = j * 16 + lane
            v = plsc.load_gather(sv, [idxv])
            m = v > _CONF_THRESH
            k = plsc.bitcast(v, jnp.int32)
            cm = jnp.logical_and(m, k >= lo)
            cmi = cm.astype(jnp.int32)
            pos = off + plsc.cumsum(cmi) - 1
            guard = jnp.logical_and(cm, pos < _CCAP)
            plsc.store_scatter(cb_key, [pos], k, mask=guard)
            plsc.store_scatter(cb_idx, [pos], idxv, mask=guard)
            return jnp.minimum(off + jnp.sum(cmi), _CCAP)
        def coll4(i, off):
            for u in range(4):
                off = coll(i * 4 + u, off)
            return off
        off = lax.fori_loop(0, _NVEC // 4, coll4, jnp.int32(0))

        @pl.when(off <= 128)
        def _small():
            bitonic(cb_key, cb_idx, 128)

        @pl.when(off > 128)
        def _big():
            bitonic(cb_key, cb_idx, _CBUF)

        for j in range(_CAND // 16):
            iv = plsc.load_gather(cb_idx, [j * 16 + lane])
            plsc.store_scatter(outp, [j * 16 + lane], iv + c * _PADN)
        pltpu.async_copy(payload_hbm.at[outp], cand_rows, sem).wait()

        def getcol(j):
            r = j * 16 + lane
            x1 = plsc.load_gather(cand_rows, [r, jnp.full((16,), 1, jnp.int32)])
            y1 = plsc.load_gather(cand_rows, [r, jnp.full((16,), 2, jnp.int32)])
            x2 = plsc.load_gather(cand_rows, [r, jnp.full((16,), 3, jnp.int32)])
            y2 = plsc.load_gather(cand_rows, [r, jnp.full((16,), 4, jnp.int32)])
            plsc.store_scatter(bx, [0 * _CAND + r], x1)
            plsc.store_scatter(bx, [1 * _CAND + r], y1)
            plsc.store_scatter(bx, [2 * _CAND + r], x2)
            plsc.store_scatter(bx, [3 * _CAND + r], y2)
            plsc.store_scatter(bx, [4 * _CAND + r], (x2 - x1) * (y2 - y1))
            plsc.store_scatter(keep, [r], zero16)
        for j in range(_CAND // 16):
            getcol(j)

        nch = _CAND // 16
        bxv = []
        for j in range(nch):
            r = j * 16 + lane
            bxv.append(tuple(plsc.load_gather(bx, [k * _CAND + r])
                             for k in range(5)))

        def nms_step(i, kcarry):
            i16 = jnp.full((16,), i, jnp.int32)
            xx1 = plsc.load_gather(bx, [0 * _CAND + i16])
            yy1 = plsc.load_gather(bx, [1 * _CAND + i16])
            xx2 = plsc.load_gather(bx, [2 * _CAND + i16])
            yy2 = plsc.load_gather(bx, [3 * _CAND + i16])
            aar = plsc.load_gather(bx, [4 * _CAND + i16])
            key_i = plsc.load_gather(cb_key, [i16])[0]
            ichunk = lax.shift_right_logical(i, 4)
            iline = jnp.bitwise_and(i, 15)
            sup = jnp.zeros((16,), jnp.bool_)
            for j in range(nch):
                x1, y1, x2, y2, ar = bxv[j]
                iw = jnp.maximum(jnp.minimum(x2, xx2) - jnp.maximum(x1, xx1),
                                 0.0)
                ih = jnp.maximum(jnp.minimum(y2, yy2) - jnp.maximum(y1, yy1),
                                 0.0)
                inter = iw * ih
                iou = inter / (ar + aar - inter)
                hit = jnp.logical_and(kcarry[j], iou > _NMS_THRESH)
                sup = jnp.logical_or(sup, hit)
            kv = jnp.logical_and(key_i > _BITS001,
                                 jnp.logical_not(jnp.any(sup)))
            hitlane = lane == iline
            return tuple(
                jnp.logical_or(kcarry[j],
                               jnp.logical_and(jnp.logical_and(hitlane, kv),
                                               ichunk == j))
                for j in range(nch))

        kfin = lax.fori_loop(0, _TOPK, nms_step,
                             tuple(jnp.zeros((16,), jnp.bool_)
                                   for _ in range(nch)))
        for j in range(nch):
            plsc.store_scatter(keep, [j * 16 + lane],
                               kfin[j].astype(jnp.int32))

        pltpu.sync_copy(cb_idx.at[pl.ds(0, _CAND)], shared.at[s, 0])
        pltpu.sync_copy(cb_key.at[pl.ds(0, _CAND)], shared.at[s, 1])
        pltpu.sync_copy(keep, shared.at[s, 2])

    plsc.subcore_barrier()

    @pl.when(s == _NCL)
    def _phase2():
        for u in range(_NCL):
            pltpu.sync_copy(shared.at[u, 0], m_prior.at[pl.ds(u * _CAND, _CAND)])
            pltpu.sync_copy(shared.at[u, 1], m_key.at[pl.ds(u * _CAND, _CAND)])
            pltpu.sync_copy(shared.at[u, 2], m_keep.at[pl.ds(u * _CAND, _CAND)])
        unrolled(_NVEC, 4,
                 lambda j: plsc.store_scatter(seen, [j * 16 + lane], zero16))
        for u in range(_NCL):
            for j in range(_CAND // 16):
                sl = u * _CAND + j * 16 + lane
                p = plsc.load_gather(m_prior, [sl])
                kp = plsc.load_gather(m_keep, [sl]) > 0
                dup = plsc.load_gather(seen, [p]) > 0
                k = plsc.load_gather(m_key, [sl])
                newk = jnp.where(
                    jnp.logical_and(kp, jnp.logical_not(dup)), k, 0)
                plsc.store_scatter(m_key, [sl], newk)
                plsc.store_scatter(seen, [p], ones16, mask=kp)
        zrow16 = jnp.full((16,), _ZROW, jnp.int32)
        for j in range(16):
            plsc.store_scatter(cb_key, [j * 16 + lane], zero16)
            plsc.store_scatter(cb_idx, [j * 16 + lane], zrow16)

        def compress_class(u, reverse):
            off = jnp.int32(0)
            for j in range(_CAND // 16):
                sl = u * _CAND + j * 16 + lane
                k = plsc.load_gather(m_key, [sl])
                p = plsc.load_gather(m_prior, [sl])
                m = k > _BITS001
                mi = m.astype(jnp.int32)
                pos = off + plsc.cumsum(mi) - 1
                tgt = (255 - pos) if reverse else pos
                plsc.store_scatter(cb_key, [tgt], k, mask=m)
                plsc.store_scatter(cb_idx, [tgt], p, mask=m)
                off = off + jnp.sum(mi)

        compress_class(0, False)
        for u in range(1, _NCL):
            if u > 1:
                for j in range(8, 16):
                    plsc.store_scatter(cb_key, [j * 16 + lane], zero16)
                    plsc.store_scatter(cb_idx, [j * 16 + lane], zrow16)
            compress_class(u, True)
            for j16 in (8, 4, 2, 1):
                cross_sweep(cb_key, cb_idx, 16, j16, 16)
            vsort_sweep(cb_key, cb_idx, 16, 16)
        for j in range(_CAND // 16):
            r = j * 16 + lane
            k = plsc.load_gather(cb_key, [r])
            p = plsc.load_gather(cb_idx, [r])
            o = jnp.where(k > _BITS001, p, _ZROW) + c * _PADN
            plsc.store_scatter(outp, [r], o)
        pltpu.async_copy(payload_hbm.at[outp], cand_rows, sem).wait()
        pltpu.sync_copy(cand_rows, out_hbm.at[c])


def _sc_select(scores, payload):
    mesh = plsc.VectorSubcoreMesh(core_axis_name="c", subcore_axis_name="s",
                                  num_cores=_B, num_subcores=16)
    f = pl.kernel(
        _sc_select_body,
        out_type=jax.ShapeDtypeStruct((_B, _CAND, _PCOLS), jnp.float32),
        mesh=mesh,
        compiler_params=pltpu.CompilerParams(needs_layout_passes=False),
        scratch_types=[
            pltpu.VMEM((_PADN,), jnp.float32),
            pltpu.VMEM((_CBUF,), jnp.int32),
            pltpu.VMEM((_CBUF,), jnp.int32),
            pltpu.VMEM((4096,), jnp.int32),
            pltpu.VMEM((_CAND, _PCOLS), jnp.float32),
            pltpu.VMEM((5 * _CAND,), jnp.float32),
            pltpu.VMEM((_CAND,), jnp.int32),
            pltpu.VMEM((_MRG,), jnp.int32),
            pltpu.VMEM((_MRG,), jnp.int32),
            pltpu.VMEM((_MRG,), jnp.int32),
            pltpu.VMEM((_PADN,), jnp.int32),
            pltpu.VMEM((_CAND,), jnp.int32),
            pltpu.VMEM_SHARED((16, 3, _CAND), jnp.int32),
            pltpu.SemaphoreType.DMA,
        ],
    )
    return f(scores.reshape(-1), payload)


def _select_jax(scores, payload):
    outs = []
    for i in range(_B):
        keys_all, prior_all, kept_all = [], [], []
        for c in range(_NCL):
            s = scores[i * _NCL + c]
            masked = jnp.where(s > _CONF_THRESH, s, -jnp.inf)
            _, cand = lax.top_k(masked, _TOPK)
            sv = s[cand]
            valid = sv > _CONF_THRESH
            rows = payload[i * _PADN + cand]
            x1, y1, x2, y2 = rows[:, 1], rows[:, 2], rows[:, 3], rows[:, 4]
            area = (x2 - x1) * (y2 - y1)
            ix1 = jnp.maximum(x1[:, None], x1[None, :])
            iy1 = jnp.maximum(y1[:, None], y1[None, :])
            ix2 = jnp.minimum(x2[:, None], x2[None, :])
            iy2 = jnp.minimum(y2[:, None], y2[None, :])
            iw = jnp.clip(ix2 - ix1, 0.0, None)
            ih = jnp.clip(iy2 - iy1, 0.0, None)
            inter = iw * ih
            iou = inter / (area[:, None] + area[None, :] - inter)
            keep = jnp.zeros((_TOPK,), bool)
            def nms_step(k, keep):
                sup = jnp.any(keep & (iou[:, k] > _NMS_THRESH))
                return keep.at[k].set(valid[k] & jnp.logical_not(sup))
            keep = lax.fori_loop(0, _TOPK, nms_step, keep)
            keys_all.append(jnp.where(keep, sv, 0.0))
            prior_all.append(cand)
            kept_all.append(keep)
        keys = jnp.concatenate(keys_all)
        prior = jnp.concatenate(prior_all)
        kept = jnp.concatenate(kept_all)
        M = keys.shape[0]
        same = prior[:, None] == prior[None, :]
        earlier = jnp.arange(M)[None, :] < jnp.arange(M)[:, None]
        dup = jnp.any(same & earlier & kept[None, :], axis=1)
        final = kept & jnp.logical_not(dup)
        key_bits = jnp.where(final, keys, 0.0)
        order = jnp.argsort(-key_bits)[:_TOPK]
        sel_prior = jnp.where(key_bits[order] > _CONF_THRESH,
                              prior[order], _ZROW)
        outs.append(payload[i * _PADN + sel_prior])
    return jnp.stack(outs)


def kernel(loc_data, conf_data, prior_data, targets):
    scores, payload = _tc_prep(loc_data, conf_data, prior_data, targets)
    result = _sc_select(scores, payload)[:, :_TOPK, :]
    rois = result[..., 0:5]
    loc = result[..., 5:9]
    cls = result[..., 9:9 + _C]
    loc_truth = result[..., 9 + _C:13 + _C]
    conf_truth = result[..., 13 + _C:14 + _C]
    return rois, loc, cls, loc_truth, conf_truth


# speedup vs baseline: 42.2753x; 1.0164x over previous
import functools
import struct

import jax
import jax.numpy as jnp
from jax import lax
from jax.experimental import pallas as pl
from jax.experimental.pallas import tpu as pltpu
from jax.experimental.pallas import tpu_sc as plsc

_C = 6
_NCL = _C - 1
_B = 2
_NP = 5000
_PADN = 5120
_TOPK = 100
_CAND = 112
_OVERLAP_THRESH = 0.5
_CONF_THRESH = 0.01
_NMS_THRESH = 0.45
_VAR0 = 0.1
_VAR1 = 0.2
_NOBJ = 8
_COLS = 24
_PCOLS = 128
_ZROW = _NP
_BITS001 = struct.unpack("<i", struct.pack("<f", _CONF_THRESH))[0]


def _tc_prep_body(locT_ref, confT_ref, priorsT_ref, targets_ref,
                  scores_ref, payload_ref):
    cx = priorsT_ref[0:1, :]
    cy = priorsT_ref[1:2, :]
    pw = priorsT_ref[2:3, :]
    ph = priorsT_ref[3:4, :]
    col = lax.broadcasted_iota(jnp.int32, (1, _PADN), 1)
    real = col < _NP
    realf = real.astype(jnp.float32)
    px1 = cx - pw * 0.5
    py1 = cy - ph * 0.5
    px2 = cx + pw * 0.5
    py2 = cy + ph * 0.5
    p_area = (px2 - px1) * (py2 - py1)

    for i in range(_B):
        tx1 = targets_ref[i, :, 0:1]
        ty1 = targets_ref[i, :, 1:2]
        tx2 = targets_ref[i, :, 2:3]
        ty2 = targets_ref[i, :, 3:4]
        tlab = targets_ref[i, :, 4:5]
        t_area = (tx2 - tx1) * (ty2 - ty1)
        ix1 = jnp.maximum(tx1, px1)
        iy1 = jnp.maximum(ty1, py1)
        ix2 = jnp.minimum(tx2, px2)
        iy2 = jnp.minimum(ty2, py2)
        iw = jnp.clip(ix2 - ix1, 0.0, None)
        ih = jnp.clip(iy2 - iy1, 0.0, None)
        inter = iw * ih
        ov = inter / (t_area + p_area - inter)
        ov = jnp.where(real, ov, -1.0)
        bigcol = jnp.where(ov == jnp.max(ov, axis=1, keepdims=True),
                           jnp.broadcast_to(col, ov.shape), _PADN)
        best_prior_idx = jnp.min(bigcol, axis=1, keepdims=True)
        trow = lax.broadcasted_iota(jnp.int32, (_NOBJ, 1), 0)
        ovmax0 = jnp.max(ov, axis=0, keepdims=True)
        bigrow = jnp.where(ov == ovmax0,
                           jnp.broadcast_to(trow, ov.shape), _NOBJ)
        best_truth_idx = jnp.min(bigrow, axis=0, keepdims=True)
        best_truth_overlap = ovmax0
        for t in range(_NOBJ):
            hit = col == best_prior_idx[t, 0]
            best_truth_overlap = jnp.where(hit, 2.0, best_truth_overlap)
            best_truth_idx = jnp.where(hit, t, best_truth_idx)
        mx1 = jnp.zeros((1, _PADN), jnp.float32)
        my1 = jnp.zeros((1, _PADN), jnp.float32)
        mx2 = jnp.zeros((1, _PADN), jnp.float32)
        my2 = jnp.zeros((1, _PADN), jnp.float32)
        mlab = jnp.zeros((1, _PADN), jnp.float32)
        for t in range(_NOBJ):
            sel = best_truth_idx == t
            mx1 = jnp.where(sel, tx1[t, 0], mx1)
            my1 = jnp.where(sel, ty1[t, 0], my1)
            mx2 = jnp.where(sel, tx2[t, 0], mx2)
            my2 = jnp.where(sel, ty2[t, 0], my2)
            mlab = jnp.where(sel, tlab[t, 0], mlab)
        conf_t = jnp.where(best_truth_overlap < _OVERLAP_THRESH, 0.0,
                           mlab + 1.0)
        g_cx = ((mx1 + mx2) * 0.5 - cx) / (_VAR0 * pw)
        g_cy = ((my1 + my2) * 0.5 - cy) / (_VAR0 * ph)
        safe_w = jnp.where(real, (mx2 - mx1) / pw, 1.0)
        safe_h = jnp.where(real, (my2 - my1) / ph, 1.0)
        g_w = jnp.log(safe_w) / _VAR1
        g_h = jnp.log(safe_h) / _VAR1

        l0 = locT_ref[i, 0:1, :]
        l1 = locT_ref[i, 1:2, :]
        l2 = locT_ref[i, 2:3, :]
        l3 = locT_ref[i, 3:4, :]
        dcx = cx + l0 * _VAR0 * pw
        dcy = cy + l1 * _VAR0 * ph
        dw = pw * jnp.exp(l2 * _VAR1)
        dh = ph * jnp.exp(l3 * _VAR1)
        dx1 = dcx - dw * 0.5
        dy1 = dcy - dh * 0.5
        dx2 = dx1 + dw
        dy2 = dy1 + dh

        for c in range(1, _C):
            srow = confT_ref[i, c:c + 1, :] * realf
            scores_ref[i * _NCL + (c - 1), :] = srow[0]

        cols = [jnp.full((1, _PADN), float(i), jnp.float32),
                dx1, dy1, dx2, dy2,
                l0, l1, l2, l3]
        for c in range(_C):
            cols.append(confT_ref[i, c:c + 1, :])
        cols += [g_cx, g_cy, g_w, g_h, conf_t]
        while len(cols) < _COLS:
            cols.append(jnp.zeros((1, _PADN), jnp.float32))
        tab = jnp.concatenate([v * realf for v in cols], axis=0)
        payload_ref[i, :, 0:_COLS] = jnp.transpose(tab, (1, 0))
        payload_ref[i, :, _COLS:_PCOLS] = jnp.zeros(
            (_PADN, _PCOLS - _COLS), jnp.float32)


def _tc_prep(loc_data, conf_data, prior_data, targets):
    locT = jnp.pad(jnp.transpose(loc_data, (0, 2, 1)),
                   ((0, 0), (0, 0), (0, _PADN - _NP)))
    conf3 = conf_data.reshape(_B, _NP, _C)
    confT = jnp.pad(jnp.transpose(conf3, (0, 2, 1)),
                    ((0, 0), (0, 0), (0, _PADN - _NP)))
    priorsT = jnp.pad(prior_data.T, ((0, 0), (0, _PADN - _NP)),
                      constant_values=1.0)
    padmask = jnp.arange(_PADN)[None, :] >= _NP
    priorsT = jnp.where(padmask & (jnp.arange(4)[:, None] < 2), 100.0, priorsT)
    scores, payload = pl.pallas_call(
        _tc_prep_body,
        out_shape=[
            jax.ShapeDtypeStruct((_B * _NCL, _PADN), jnp.float32),
            jax.ShapeDtypeStruct((_B, _PADN, _PCOLS), jnp.float32),
        ],
    )(locT, confT, priorsT, targets)
    return scores, payload.reshape(_B * _PADN, _PCOLS)


_CBUF = 512
_CCAP = _CBUF - 16
_MRG = 1024
_NVEC = _PADN // 16


def _sc_select_body(scores_hbm, payload_hbm, out_hbm,
                    sv, cb_key, cb_idx, hist, cand_rows, bx, keep,
                    m_all, pub, seen, outp, shared, sem):
    c = lax.axis_index("c")
    s = lax.axis_index("s")
    lane = lax.iota(jnp.int32, 16)
    ones16 = jnp.ones((16,), jnp.int32)
    zero16 = jnp.zeros((16,), jnp.int32)

    def unrolled(n, unroll, fn):
        def body(i, _):
            for u in range(unroll):
                fn(i * unroll + u)
            return 0
        lax.fori_loop(0, n // unroll, body, 0)
        for j in range(n - n % unroll, n):
            fn(j)

    def hist_zero():
        unrolled(256, 4,
                 lambda j: plsc.store_scatter(hist, [j * 16 + lane], zero16))

    def hist_pass(bucket_fn):
        def body(j):
            b, m = bucket_fn(j)
            plsc.addupdate_scatter(hist, [lane * 256 + b], ones16, mask=m)
        unrolled(_NVEC, 4, body)

    def hist_select(rank):
        run = jnp.int32(0)
        B = jnp.int32(0)
        ca = jnp.int32(0)
        found = jnp.bool_(False)
        for g in range(15, -1, -1):
            acc = zero16
            for l in range(16):
                acc = acc + hist[pl.ds(l * 256 + g * 16, 16)]
            sfx = lax.rev(plsc.cumsum(lax.rev(acc, (0,))), (0,)) + run
            mask = sfx >= rank
            cnt = plsc.all_reduce_population_count(mask)[0]
            cav = jnp.sum(jnp.where(lane == cnt, sfx, 0))
            ca_g = jnp.where(cnt == 16, run, cav)
            hit = jnp.logical_and(jnp.logical_not(found), cnt > 0)
            B = jnp.where(hit, g * 16 + cnt - 1, B)
            ca = jnp.where(hit, ca_g, ca)
            found = jnp.logical_or(found, cnt > 0)
            run = run + jnp.sum(acc)
        return B, ca, found

    def vsort_sweep(keyref, valref, nb, k16):
        def body(b, _):
            base = b * 16
            key = plsc.load_gather(keyref, [base + lane])
            val = plsc.load_gather(valref, [base + lane])
            desc = (b & k16) == 0
            tkey = jnp.where(desc, key, -1 - key)
            skey, sval = plsc.sort_key_val(tkey, val, descending=True)
            skey = jnp.where(desc, skey, -1 - skey)
            plsc.store_scatter(keyref, [base + lane], skey)
            plsc.store_scatter(valref, [base + lane], sval)
            return 0
        lax.fori_loop(0, nb, body, 0)

    def cross_sweep(keyref, valref, nb, j16, k16):
        p = j16.bit_length() - 1
        def body(t, _):
            b = lax.shift_left(lax.shift_right_logical(t, p), p + 1) \
                + jnp.bitwise_and(t, j16 - 1)
            base_a = b * 16
            base_b = (b + j16) * 16
            ka = plsc.load_gather(keyref, [base_a + lane])
            va = plsc.load_gather(valref, [base_a + lane])
            kb = plsc.load_gather(keyref, [base_b + lane])
            vb = plsc.load_gather(valref, [base_b + lane])
            desc = (b & k16) == 0
            swap = jnp.where(desc, ka < kb, ka > kb)
            plsc.store_scatter(keyref, [base_a + lane],
                               jnp.where(swap, kb, ka))
            plsc.store_scatter(keyref, [base_b + lane],
                               jnp.where(swap, ka, kb))
            plsc.store_scatter(valref, [base_a + lane],
                               jnp.where(swap, vb, va))
            plsc.store_scatter(valref, [base_b + lane],
                               jnp.where(swap, va, vb))
            return 0
        lax.fori_loop(0, nb // 2, body, 0)

    def bitonic(keyref, valref, n):
        nb = n // 16
        vsort_sweep(keyref, valref, nb, 1)
        kk = 32
        while kk <= n:
            j = kk // 2
            while j >= 16:
                cross_sweep(keyref, valref, nb, j // 16, kk // 16)
                j //= 2
            vsort_sweep(keyref, valref, nb, kk // 16)
            kk *= 2

    @pl.when(s < _NCL)
    def _phase1():
        row = c * _NCL + s
        pltpu.sync_copy(scores_hbm.at[pl.ds(row * _PADN, _PADN)], sv)

        def load_chunk(j):
            v = plsc.load_gather(sv, [j * 16 + lane])
            m = v > _CONF_THRESH
            k = plsc.bitcast(v, jnp.int32)
            return v, m, k

        hist_zero()
        def bucket_a(j):
            _, m, k = load_chunk(j)
            return lax.shift_right_logical(k, 23), m
        hist_pass(bucket_a)
        b1, ca1, found1 = hist_select(jnp.int32(_TOPK))

        hist_zero()
        def bucket_b(j):
            _, m, k = load_chunk(j)
            m2 = jnp.logical_and(m, lax.shift_right_logical(k, 23) == b1)
            return jnp.bitwise_and(lax.shift_right_logical(k, 15), 255), m2
        hist_pass(bucket_b)
        b2, _, _ = hist_select(_TOPK - ca1)
        lo = jnp.where(found1,
                       jnp.bitwise_or(lax.shift_left(b1, 23),
                                      lax.shift_left(b2, 15)),
                       jnp.int32(0))

        def initcb(j):
            plsc.store_scatter(cb_key, [j * 16 + lane], zero16)
            plsc.store_scatter(cb_idx, [j * 16 + lane],
                               jnp.full((16,), _ZROW, jnp.int32))
        unrolled(_CBUF // 16, 4, initcb)

        def coll(j, off):
            idxv = j * 16 + lane
            v = plsc.load_gather(sv, [idxv])
            m = v > _CONF_THRESH
            k = plsc.bitcast(v, jnp.int32)
            cm = jnp.logical_and(m, k >= lo)
            cmi = cm.astype(jnp.int32)
            pos = off + plsc.cumsum(cmi) - 1
            guard = jnp.logical_and(cm, pos < _CCAP)
            plsc.store_scatter(cb_key, [pos], k, mask=guard)
            plsc.store_scatter(cb_idx, [pos], idxv, mask=guard)
            return jnp.minimum(off + jnp.sum(cmi), _CCAP)
        def coll4(i, off):
            for u in range(4):
                off = coll(i * 4 + u, off)
            return off
        off = lax.fori_loop(0, _NVEC // 4, coll4, jnp.int32(0))

        @pl.when(off <= 128)
        def _small():
            bitonic(cb_key, cb_idx, 128)

        @pl.when(off > 128)
        def _big():
            bitonic(cb_key, cb_idx, _CBUF)

        for j in range(_CAND // 16):
            iv = plsc.load_gather(cb_idx, [j * 16 + lane])
            plsc.store_scatter(outp, [j * 16 + lane], iv + c * _PADN)
        pltpu.async_copy(payload_hbm.at[outp], cand_rows, sem).wait()

        def getcol(j):
            r = j * 16 + lane
            x1 = plsc.load_gather(cand_rows, [r, jnp.full((16,), 1, jnp.int32)])
            y1 = plsc.load_gather(cand_rows, [r, jnp.full((16,), 2, jnp.int32)])
            x2 = plsc.load_gather(cand_rows, [r, jnp.full((16,), 3, jnp.int32)])
            y2 = plsc.load_gather(cand_rows, [r, jnp.full((16,), 4, jnp.int32)])
            plsc.store_scatter(bx, [0 * _CAND + r], x1)
            plsc.store_scatter(bx, [1 * _CAND + r], y1)
            plsc.store_scatter(bx, [2 * _CAND + r], x2)
            plsc.store_scatter(bx, [3 * _CAND + r], y2)
            plsc.store_scatter(bx, [4 * _CAND + r], (x2 - x1) * (y2 - y1))
            plsc.store_scatter(keep, [r], zero16)
        for j in range(_CAND // 16):
            getcol(j)

        nch = _CAND // 16
        bxv = []
        for j in range(nch):
            r = j * 16 + lane
            bxv.append(tuple(plsc.load_gather(bx, [k * _CAND + r])
                             for k in range(5)))

        def nms_step(i, kcarry):
            i16 = jnp.full((16,), i, jnp.int32)
            xx1 = plsc.load_gather(bx, [0 * _CAND + i16])
            yy1 = plsc.load_gather(bx, [1 * _CAND + i16])
            xx2 = plsc.load_gather(bx, [2 * _CAND + i16])
            yy2 = plsc.load_gather(bx, [3 * _CAND + i16])
            aar = plsc.load_gather(bx, [4 * _CAND + i16])
            key_i = plsc.load_gather(cb_key, [i16])[0]
            ichunk = lax.shift_right_logical(i, 4)
            iline = jnp.bitwise_and(i, 15)
            sup = jnp.zeros((16,), jnp.bool_)
            for j in range(nch):
                x1, y1, x2, y2, ar = bxv[j]
                iw = jnp.maximum(jnp.minimum(x2, xx2) - jnp.maximum(x1, xx1),
                                 0.0)
                ih = jnp.maximum(jnp.minimum(y2, yy2) - jnp.maximum(y1, yy1),
                                 0.0)
                inter = iw * ih
                iou = inter / (ar + aar - inter)
                hit = jnp.logical_and(kcarry[j], iou > _NMS_THRESH)
                sup = jnp.logical_or(sup, hit)
            kv = jnp.logical_and(key_i > _BITS001,
                                 jnp.logical_not(jnp.any(sup)))
            hitlane = lane == iline
            return tuple(
                jnp.logical_or(kcarry[j],
                               jnp.logical_and(jnp.logical_and(hitlane, kv),
                                               ichunk == j))
                for j in range(nch))

        kfin = lax.fori_loop(0, _TOPK, nms_step,
                             tuple(jnp.zeros((16,), jnp.bool_)
                                   for _ in range(nch)))
        for j in range(nch):
            plsc.store_scatter(keep, [j * 16 + lane],
                               kfin[j].astype(jnp.int32))

        for j in range(_CAND // 16):
            r = j * 16 + lane
            plsc.store_scatter(pub, [r], plsc.load_gather(cb_idx, [r]))
            plsc.store_scatter(pub, [_CAND + r],
                               plsc.load_gather(cb_key, [r]))
            plsc.store_scatter(pub, [2 * _CAND + r],
                               plsc.load_gather(keep, [r]))
        pltpu.sync_copy(pub, shared.at[s])

    plsc.subcore_barrier()

    @pl.when(s == _NCL)
    def _phase2():
        pltpu.sync_copy(shared.at[pl.ds(0, _NCL)], m_all)
        unrolled(_NVEC, 4,
                 lambda j: plsc.store_scatter(seen, [j * 16 + lane], zero16))
        def fl(v):
            return jnp.full((16,), v, jnp.int32)
        for u in range(_NCL):
            for j in range(_CAND // 16):
                r = j * 16 + lane
                p = plsc.load_gather(m_all, [fl(u), r])
                kp = plsc.load_gather(m_all, [fl(u), 2 * _CAND + r]) > 0
                dup = plsc.load_gather(seen, [p]) > 0
                k = plsc.load_gather(m_all, [fl(u), _CAND + r])
                newk = jnp.where(
                    jnp.logical_and(kp, jnp.logical_not(dup)), k, 0)
                plsc.store_scatter(m_all, [fl(u), _CAND + r], newk)
                plsc.store_scatter(seen, [p], ones16, mask=kp)
        zrow16 = jnp.full((16,), _ZROW, jnp.int32)
        for j in range(16):
            plsc.store_scatter(cb_key, [j * 16 + lane], zero16)
            plsc.store_scatter(cb_idx, [j * 16 + lane], zrow16)

        def compress_class(u, reverse):
            off = jnp.int32(0)
            for j in range(_CAND // 16):
                r = j * 16 + lane
                k = plsc.load_gather(m_all, [fl(u), _CAND + r])
                p = plsc.load_gather(m_all, [fl(u), r])
                m = k > _BITS001
                mi = m.astype(jnp.int32)
                pos = off + plsc.cumsum(mi) - 1
                tgt = (255 - pos) if reverse else pos
                plsc.store_scatter(cb_key, [tgt], k, mask=m)
                plsc.store_scatter(cb_idx, [tgt], p, mask=m)
                off = off + jnp.sum(mi)

        compress_class(0, False)
        for u in range(1, _NCL):
            if u > 1:
                for j in range(8, 16):
                    plsc.store_scatter(cb_key, [j * 16 + lane], zero16)
                    plsc.store_scatter(cb_idx, [j * 16 + lane], zrow16)
            compress_class(u, True)
            for j16 in (8, 4, 2, 1):
                cross_sweep(cb_key, cb_idx, 16, j16, 16)
            vsort_sweep(cb_key, cb_idx, 16, 16)
        for j in range(_CAND // 16):
            r = j * 16 + lane
            k = plsc.load_gather(cb_key, [r])
            p = plsc.load_gather(cb_idx, [r])
            o = jnp.where(k > _BITS001, p, _ZROW) + c * _PADN
            plsc.store_scatter(outp, [r], o)
        pltpu.async_copy(payload_hbm.at[outp], cand_rows, sem).wait()
        pltpu.sync_copy(cand_rows, out_hbm.at[c])


def _sc_select(scores, payload):
    mesh = plsc.VectorSubcoreMesh(core_axis_name="c", subcore_axis_name="s",
                                  num_cores=_B, num_subcores=16)
    f = pl.kernel(
        _sc_select_body,
        out_type=jax.ShapeDtypeStruct((_B, _CAND, _PCOLS), jnp.float32),
        mesh=mesh,
        compiler_params=pltpu.CompilerParams(needs_layout_passes=False),
        scratch_types=[
            pltpu.VMEM((_PADN,), jnp.float32),
            pltpu.VMEM((_CBUF,), jnp.int32),
            pltpu.VMEM((_CBUF,), jnp.int32),
            pltpu.VMEM((4096,), jnp.int32),
            pltpu.VMEM((_CAND, _PCOLS), jnp.float32),
            pltpu.VMEM((5 * _CAND,), jnp.float32),
            pltpu.VMEM((_CAND,), jnp.int32),
            pltpu.VMEM((_NCL, 3 * _CAND), jnp.int32),
            pltpu.VMEM((3 * _CAND,), jnp.int32),
            pltpu.VMEM((_PADN,), jnp.int32),
            pltpu.VMEM((_CAND,), jnp.int32),
            pltpu.VMEM_SHARED((16, 3 * _CAND), jnp.int32),
            pltpu.SemaphoreType.DMA,
        ],
    )
    return f(scores.reshape(-1), payload)


def _select_jax(scores, payload):
    outs = []
    for i in range(_B):
        keys_all, prior_all, kept_all = [], [], []
        for c in range(_NCL):
            s = scores[i * _NCL + c]
            masked = jnp.where(s > _CONF_THRESH, s, -jnp.inf)
            _, cand = lax.top_k(masked, _TOPK)
            sv = s[cand]
            valid = sv > _CONF_THRESH
            rows = payload[i * _PADN + cand]
            x1, y1, x2, y2 = rows[:, 1], rows[:, 2], rows[:, 3], rows[:, 4]
            area = (x2 - x1) * (y2 - y1)
            ix1 = jnp.maximum(x1[:, None], x1[None, :])
            iy1 = jnp.maximum(y1[:, None], y1[None, :])
            ix2 = jnp.minimum(x2[:, None], x2[None, :])
            iy2 = jnp.minimum(y2[:, None], y2[None, :])
            iw = jnp.clip(ix2 - ix1, 0.0, None)
            ih = jnp.clip(iy2 - iy1, 0.0, None)
            inter = iw * ih
            iou = inter / (area[:, None] + area[None, :] - inter)
            keep = jnp.zeros((_TOPK,), bool)
            def nms_step(k, keep):
                sup = jnp.any(keep & (iou[:, k] > _NMS_THRESH))
                return keep.at[k].set(valid[k] & jnp.logical_not(sup))
            keep = lax.fori_loop(0, _TOPK, nms_step, keep)
            keys_all.append(jnp.where(keep, sv, 0.0))
            prior_all.append(cand)
            kept_all.append(keep)
        keys = jnp.concatenate(keys_all)
        prior = jnp.concatenate(prior_all)
        kept = jnp.concatenate(kept_all)
        M = keys.shape[0]
        same = prior[:, None] == prior[None, :]
        earlier = jnp.arange(M)[None, :] < jnp.arange(M)[:, None]
        dup = jnp.any(same & earlier & kept[None, :], axis=1)
        final = kept & jnp.logical_not(dup)
        key_bits = jnp.where(final, keys, 0.0)
        order = jnp.argsort(-key_bits)[:_TOPK]
        sel_prior = jnp.where(key_bits[order] > _CONF_THRESH,
                              prior[order], _ZROW)
        outs.append(payload[i * _PADN + sel_prior])
    return jnp.stack(outs)


def kernel(loc_data, conf_data, prior_data, targets):
    scores, payload = _tc_prep(loc_data, conf_data, prior_data, targets)
    result = _sc_select(scores, payload)[:, :_TOPK, :]
    rois = result[..., 0:5]
    loc = result[..., 5:9]
    cls = result[..., 9:9 + _C]
    loc_truth = result[..., 9 + _C:13 + _C]
    conf_truth = result[..., 13 + _C:14 + _C]
    return rois, loc, cls, loc_truth, conf_truth
